# Initial kernel scaffold; baseline (speedup 1.0000x reference)
import functools

import jax
import jax.numpy as jnp
from jax import lax
from jax.experimental import pallas as pl
from jax.experimental.pallas import tpu as pltpu
from jax.experimental.pallas import tpu_sc as plsc

_EMA = 0.1
_EPS = 1e-08
_NEG_INF = float("-inf")
_BIG_I32 = jnp.int32(2 ** 30)


def _topk_body(x_ref, wa_ref, a_ref, idx_ref, w_ref, q_scr, tv_scr, ti_scr):
    m = pl.program_id(1)
    n_m = pl.num_programs(1)
    nt, mt = tv_scr.shape[0], a_ref.shape[0]

    @pl.when(m == 0)
    def _init():
        q = lax.dot_general(x_ref[...], wa_ref[...], (((1,), (1,)), ((), ())),
                            preferred_element_type=jnp.float32)
        q_scr[...] = q
        tv_scr[...] = jnp.full((nt, 4), _NEG_INF, jnp.float32)
        ti_scr[...] = jnp.full((nt, 4), _BIG_I32, jnp.int32)

    a = a_ref[...]
    ssa = jnp.sum(a * a, axis=1)[None, :]
    scale = 1.0 / jnp.maximum(jnp.sqrt(ssa), _EPS)
    s = lax.dot_general(q_scr[...], a, (((1,), (1,)), ((), ())),
                        preferred_element_type=jnp.float32)
    s = s * scale

    iota_m = lax.broadcasted_iota(jnp.int32, (nt, mt), 1) + m * mt
    bvs, bis = [], []
    for _ in range(4):
        mx = jnp.max(s, axis=1, keepdims=True)
        ix = jnp.min(jnp.where(s == mx, iota_m, _BIG_I32), axis=1,
                     keepdims=True)
        bvs.append(mx)
        bis.append(ix)
        s = jnp.where(iota_m == ix, _NEG_INF, s)

    cv = jnp.concatenate([tv_scr[...]] + bvs, axis=1)
    ci = jnp.concatenate([ti_scr[...]] + bis, axis=1)
    nvs, nis = [], []
    for _ in range(4):
        mx = jnp.max(cv, axis=1, keepdims=True)
        ix = jnp.min(jnp.where(cv == mx, ci, _BIG_I32), axis=1,
                     keepdims=True)
        nvs.append(mx)
        nis.append(ix)
        cv = jnp.where(ci == ix, _NEG_INF, cv)
    tv_scr[...] = jnp.concatenate(nvs, axis=1)
    ti_scr[...] = jnp.concatenate(nis, axis=1)

    @pl.when(m == n_m - 1)
    def _final():
        q = q_scr[...]
        qn = jnp.sqrt(jnp.sum(q * q, axis=1, keepdims=True))
        tv = tv_scr[...] / jnp.maximum(qn, _EPS)
        e = jnp.exp(tv - jnp.max(tv, axis=1, keepdims=True))
        w_ref[...] = e / jnp.sum(e, axis=1, keepdims=True)
        idx_ref[...] = ti_scr[...]


def _scores_topk(xf, w_addr, addresses, nt, mt):
    n, h = xf.shape
    m_, _ = addresses.shape
    grid = (n // nt, m_ // mt)
    return pl.pallas_call(
        _topk_body,
        grid=grid,
        in_specs=[
            pl.BlockSpec((nt, h), lambda i, j: (i, 0)),
            pl.BlockSpec((h, h), lambda i, j: (0, 0)),
            pl.BlockSpec((mt, h), lambda i, j: (j, 0)),
        ],
        out_specs=[
            pl.BlockSpec((nt, 4), lambda i, j: (i, 0)),
            pl.BlockSpec((nt, 4), lambda i, j: (i, 0)),
        ],
        out_shape=[
            jax.ShapeDtypeStruct((n, 4), jnp.int32),
            jax.ShapeDtypeStruct((n, 4), jnp.float32),
        ],
        scratch_shapes=[
            pltpu.VMEM((nt, h), jnp.float32),
            pltpu.VMEM((nt, 4), jnp.float32),
            pltpu.VMEM((nt, 4), jnp.int32),
        ],
    )(xf, w_addr, addresses)


def _gather_blend(contents, idx3, w3, xf, n_workers, grp, n_grp):
    m_, h = contents.shape
    n, _ = xf.shape
    tpw = n // n_workers
    hc = h // 16
    mesh = plsc.VectorSubcoreMesh(core_axis_name="c", subcore_axis_name="s")

    @functools.partial(
        pl.kernel,
        out_type=[
            jax.ShapeDtypeStruct((n, h), jnp.float32),
            jax.ShapeDtypeStruct((n, h), jnp.float32),
        ],
        mesh=mesh,
        scratch_types=[
            pltpu.VMEM((n_grp, grp * 4), jnp.int32),
            pltpu.VMEM((n_grp, grp * 4), jnp.float32),
            pltpu.VMEM((grp * 4, h), jnp.float32),
            pltpu.VMEM((grp, h), jnp.float32),
            pltpu.VMEM((grp, h), jnp.float32),
            pltpu.VMEM((grp, h), jnp.float32),
            pltpu.SemaphoreType.DMA,
        ],
    )
    def k2(c_hbm, idx_hbm, w_hbm, x_hbm, read_hbm, upd_hbm,
           idx_v, w_v, rows_v, x_v, read_v, upd_v, sem):
        nc = lax.axis_index("c")
        ns = lax.axis_index("s")
        wid = ns * 2 + nc
        base = wid * tpw
        pltpu.sync_copy(idx_hbm.at[wid], idx_v)
        pltpu.sync_copy(w_hbm.at[wid], w_v)

        def group_body(g, carry):
            tok0 = base + g * grp
            pltpu.async_copy(c_hbm.at[idx_v.at[g]], rows_v, sem).wait()
            pltpu.sync_copy(x_hbm.at[pl.ds(tok0, grp)], x_v)

            def tok_body(t, carry2):
                gv = jnp.full((16,), g, jnp.int32)
                wk = [
                    plsc.load_gather(
                        w_v, [gv, jnp.full((16,), t * 4 + k, jnp.int32)])
                    for k in range(4)
                ]
                for hh in range(hc):
                    sl = pl.ds(hh * 16, 16)
                    r0 = rows_v[t * 4 + 0, sl]
                    r1 = rows_v[t * 4 + 1, sl]
                    r2 = rows_v[t * 4 + 2, sl]
                    r3 = rows_v[t * 4 + 3, sl]
                    read_v[t, sl] = (wk[0] * r0 + wk[1] * r1
                                     + wk[2] * r2 + wk[3] * r3)
                    upd_v[t, sl] = _EMA * (x_v[t, sl] - r0)
                return carry2

            lax.fori_loop(0, grp, tok_body, 0)
            pltpu.sync_copy(read_v, read_hbm.at[pl.ds(tok0, grp)])
            pltpu.sync_copy(upd_v, upd_hbm.at[pl.ds(tok0, grp)])
            return carry

        lax.fori_loop(0, n_grp, group_body, 0)

    return k2(contents, idx3, w3, xf)


def _proj_body(r_ref, w_ref, o_ref):
    o_ref[...] = lax.dot_general(r_ref[...], w_ref[...],
                                 (((1,), (1,)), ((), ())),
                                 preferred_element_type=jnp.float32)


def _out_proj(read, w_read, nt):
    n, h = read.shape
    return pl.pallas_call(
        _proj_body,
        grid=(n // nt,),
        in_specs=[
            pl.BlockSpec((nt, h), lambda i: (i, 0)),
            pl.BlockSpec((h, h), lambda i: (0, 0)),
        ],
        out_specs=pl.BlockSpec((nt, h), lambda i: (i, 0)),
        out_shape=jax.ShapeDtypeStruct((n, h), jnp.float32),
    )(read, w_read)


def _scatter_body(c_ref, t1_ref, u_ref, o_ref):
    nn = pl.program_id(1)
    mt, nt = o_ref.shape[0], u_ref.shape[0]
    mloc = pl.program_id(0) * mt

    @pl.when(nn == 0)
    def _init():
        o_ref[...] = c_ref[...]

    idxv = t1_ref[0, 0, :]
    miota = lax.broadcasted_iota(jnp.int32, (mt, nt), 0) + mloc
    onehot = (miota == idxv[None, :]).astype(jnp.bfloat16)
    o_ref[...] += lax.dot_general(onehot, u_ref[...],
                                  (((1,), (0,)), ((), ())),
                                  preferred_element_type=jnp.float32)


def _scatter_update(contents, top1, upd_bf, mt, nt):
    m_, h = contents.shape
    n = top1.shape[0]
    t1r = top1.reshape(n // nt, 1, nt)
    grid = (m_ // mt, n // nt)
    return pl.pallas_call(
        _scatter_body,
        grid=grid,
        in_specs=[
            pl.BlockSpec((mt, h), lambda i, j: (i, 0)),
            pl.BlockSpec((1, 1, nt), lambda i, j: (j, 0, 0)),
            pl.BlockSpec((nt, h), lambda i, j: (j, 0)),
        ],
        out_specs=pl.BlockSpec((mt, h), lambda i, j: (i, 0)),
        out_shape=jax.ShapeDtypeStruct((m_, h), jnp.float32),
    )(contents, t1r, upd_bf)


def kernel(x, addresses, contents, W_addr, W_read):
    b, s, h = x.shape
    m_, _ = addresses.shape
    n = b * s
    xf = x.reshape(n, h)

    nt1 = min(1024, n)
    mt1 = min(1024, m_)
    top_idx, w = _scores_topk(xf, W_addr, addresses, nt1, mt1)

    n_workers = 32
    grp = 16
    tpw = n // n_workers
    n_grp = tpw // grp
    idx3 = top_idx.reshape(n_workers, n_grp, grp * 4)
    w3 = w.reshape(n_workers, n_grp, grp * 4)
    read, upd = _gather_blend(contents, idx3, w3, xf, n_workers, grp, n_grp)

    out = _out_proj(read, W_read, min(1024, n)).reshape(b, s, h)

    upd_bf = upd.astype(jnp.bfloat16)
    new_contents = _scatter_update(contents, top_idx[:, 0], upd_bf,
                                   min(1024, m_), min(2048, n))
    return out, new_contents


# speedup vs baseline: 26.9000x; 26.9000x over previous
import functools

import jax
import jax.numpy as jnp
from jax import lax
from jax.experimental import pallas as pl
from jax.experimental.pallas import tpu as pltpu
from jax.experimental.pallas import tpu_sc as plsc

_EMA = 0.1
_EPS = 1e-08
_NEG_INF = float("-inf")
_BIG_I32 = 2 ** 30


def _topk_body(x_ref, wa_ref, a_ref, idx_ref, w_ref, q_scr, tv_scr, ti_scr):
    m = pl.program_id(1)
    n_m = pl.num_programs(1)
    nt, mt = tv_scr.shape[0], a_ref.shape[0]

    @pl.when(m == 0)
    def _init():
        q = lax.dot_general(x_ref[...], wa_ref[...], (((1,), (1,)), ((), ())),
                            preferred_element_type=jnp.float32)
        qn = jnp.sqrt(jnp.sum(q * q, axis=1, keepdims=True))
        q_scr[...] = (q / jnp.maximum(qn, _EPS)).astype(jnp.bfloat16)
        tv_scr[...] = jnp.full((nt, 4), _NEG_INF, jnp.float32)
        ti_scr[...] = jnp.full((nt, 4), _BIG_I32, jnp.int32)

    a = a_ref[...]
    an = jnp.sqrt(jnp.sum(a * a, axis=1, keepdims=True))
    ab = (a / jnp.maximum(an, _EPS)).astype(jnp.bfloat16)
    s = lax.dot_general(q_scr[...], ab, (((1,), (1,)), ((), ())),
                        preferred_element_type=jnp.float32)

    iota_m = lax.broadcasted_iota(jnp.int32, (nt, mt), 1) + m * mt
    bvs, bis = [], []
    for _ in range(4):
        mx = jnp.max(s, axis=1, keepdims=True)
        ix = jnp.min(jnp.where(s == mx, iota_m, _BIG_I32), axis=1,
                     keepdims=True)
        bvs.append(mx)
        bis.append(ix)
        s = jnp.where(iota_m == ix, _NEG_INF, s)

    cv = jnp.concatenate([tv_scr[...]] + bvs, axis=1)
    ci = jnp.concatenate([ti_scr[...]] + bis, axis=1)
    nvs, nis = [], []
    for _ in range(4):
        mx = jnp.max(cv, axis=1, keepdims=True)
        ix = jnp.min(jnp.where(cv == mx, ci, _BIG_I32), axis=1,
                     keepdims=True)
        nvs.append(mx)
        nis.append(ix)
        cv = jnp.where(ci == ix, _NEG_INF, cv)
    tv_scr[...] = jnp.concatenate(nvs, axis=1)
    ti_scr[...] = jnp.concatenate(nis, axis=1)

    @pl.when(m == n_m - 1)
    def _final():
        tv = tv_scr[...]
        e = jnp.exp(tv - jnp.max(tv, axis=1, keepdims=True))
        wsm = e / jnp.sum(e, axis=1, keepdims=True)
        w_ref[...] = jnp.broadcast_to(wsm[:, :, None],
                                      (nt, 4, 16)).reshape(nt, 64)
        idx_ref[...] = ti_scr[...]


def _scores_topk(xf, w_addr, addresses, nt, mt):
    n, h = xf.shape
    m_, _ = addresses.shape
    grid = (n // nt, m_ // mt)
    return pl.pallas_call(
        _topk_body,
        grid=grid,
        in_specs=[
            pl.BlockSpec((nt, h), lambda i, j: (i, 0)),
            pl.BlockSpec((h, h), lambda i, j: (0, 0)),
            pl.BlockSpec((mt, h), lambda i, j: (j, 0)),
        ],
        out_specs=[
            pl.BlockSpec((nt, 4), lambda i, j: (i, 0)),
            pl.BlockSpec((nt, 64), lambda i, j: (i, 0)),
        ],
        out_shape=[
            jax.ShapeDtypeStruct((n, 4), jnp.int32),
            jax.ShapeDtypeStruct((n, 64), jnp.float32),
        ],
        scratch_shapes=[
            pltpu.VMEM((nt, h), jnp.bfloat16),
            pltpu.VMEM((nt, 4), jnp.float32),
            pltpu.VMEM((nt, 4), jnp.int32),
        ],
    )(xf, w_addr, addresses)


def _gather_blend(contents, idx3, w3, xf, n_workers, grp, n_grp):
    m_, h = contents.shape
    n, _ = xf.shape
    tpw = n // n_workers
    hc = h // 16
    mesh = plsc.VectorSubcoreMesh(core_axis_name="c", subcore_axis_name="s")

    @functools.partial(
        pl.kernel,
        out_type=[
            jax.ShapeDtypeStruct((n, h), jnp.float32),
            jax.ShapeDtypeStruct((n, h), jnp.float32),
        ],
        mesh=mesh,
        scratch_types=[
            pltpu.VMEM((n_grp, grp * 4), jnp.int32),
            pltpu.VMEM((grp * 64,), jnp.float32),
            pltpu.VMEM((grp * 4, h), jnp.float32),
            pltpu.VMEM((grp, h), jnp.float32),
            pltpu.VMEM((grp, h), jnp.float32),
            pltpu.VMEM((grp, h), jnp.float32),
            pltpu.SemaphoreType.DMA,
        ],
    )
    def k2(c_hbm, idx_hbm, w_hbm, x_hbm, read_hbm, upd_hbm,
           idx_v, w_v, rows_v, x_v, read_v, upd_v, sem):
        nc = lax.axis_index("c")
        ns = lax.axis_index("s")
        wid = ns * 2 + nc
        base = wid * tpw
        pltpu.sync_copy(idx_hbm.at[wid], idx_v)

        def group_body(g, carry):
            tok0 = base + g * grp
            pltpu.async_copy(c_hbm.at[idx_v.at[g]], rows_v, sem).wait()
            pltpu.sync_copy(x_hbm.at[pl.ds(tok0, grp)], x_v)
            pltpu.sync_copy(w_hbm.at[wid, pl.ds(g * grp * 64, grp * 64)], w_v)

            def tok_body(t, carry2):
                wk = [w_v[pl.ds(t * 64 + k * 16, 16)] for k in range(4)]
                for hh in range(hc):
                    sl = pl.ds(hh * 16, 16)
                    r0 = rows_v[t * 4 + 0, sl]
                    r1 = rows_v[t * 4 + 1, sl]
                    r2 = rows_v[t * 4 + 2, sl]
                    r3 = rows_v[t * 4 + 3, sl]
                    read_v[t, sl] = (wk[0] * r0 + wk[1] * r1
                                     + wk[2] * r2 + wk[3] * r3)
                    upd_v[t, sl] = _EMA * (x_v[t, sl] - r0)
                return carry2

            lax.fori_loop(0, grp, tok_body, 0)
            pltpu.sync_copy(read_v, read_hbm.at[pl.ds(tok0, grp)])
            pltpu.sync_copy(upd_v, upd_hbm.at[pl.ds(tok0, grp)])
            return carry

        lax.fori_loop(0, n_grp, group_body, 0)

    return k2(contents, idx3, w3, xf)


def _proj_body(r_ref, w_ref, o_ref):
    o_ref[...] = lax.dot_general(r_ref[...], w_ref[...],
                                 (((1,), (1,)), ((), ())),
                                 preferred_element_type=jnp.float32)


def _out_proj(read, w_read, nt):
    n, h = read.shape
    return pl.pallas_call(
        _proj_body,
        grid=(n // nt,),
        in_specs=[
            pl.BlockSpec((nt, h), lambda i: (i, 0)),
            pl.BlockSpec((h, h), lambda i: (0, 0)),
        ],
        out_specs=pl.BlockSpec((nt, h), lambda i: (i, 0)),
        out_shape=jax.ShapeDtypeStruct((n, h), jnp.float32),
    )(read, w_read)


def _scatter_body(c_ref, t1_ref, u_ref, o_ref):
    nn = pl.program_id(1)
    mt, nt = o_ref.shape[0], u_ref.shape[0]
    mloc = pl.program_id(0) * mt

    @pl.when(nn == 0)
    def _init():
        o_ref[...] = c_ref[...]

    idxv = t1_ref[0, 0, :]
    miota = lax.broadcasted_iota(jnp.int32, (mt, nt), 0) + mloc
    onehot = (miota == idxv[None, :]).astype(jnp.bfloat16)
    o_ref[...] += lax.dot_general(onehot, u_ref[...],
                                  (((1,), (0,)), ((), ())),
                                  preferred_element_type=jnp.float32)


def _scatter_update(contents, top1, upd_bf, mt, nt):
    m_, h = contents.shape
    n = top1.shape[0]
    t1r = top1.reshape(n // nt, 1, nt)
    grid = (m_ // mt, n // nt)
    return pl.pallas_call(
        _scatter_body,
        grid=grid,
        in_specs=[
            pl.BlockSpec((mt, h), lambda i, j: (i, 0)),
            pl.BlockSpec((1, 1, nt), lambda i, j: (j, 0, 0)),
            pl.BlockSpec((nt, h), lambda i, j: (j, 0)),
        ],
        out_specs=pl.BlockSpec((mt, h), lambda i, j: (i, 0)),
        out_shape=jax.ShapeDtypeStruct((m_, h), jnp.float32),
    )(contents, t1r, upd_bf)


def kernel(x, addresses, contents, W_addr, W_read):
    b, s, h = x.shape
    m_, _ = addresses.shape
    n = b * s
    xf = x.reshape(n, h)

    nt1 = min(1024, n)
    mt1 = min(1024, m_)
    top_idx, w = _scores_topk(xf, W_addr, addresses, nt1, mt1)

    n_workers = 32
    grp = 16
    tpw = n // n_workers
    n_grp = tpw // grp
    idx3 = top_idx.reshape(n_workers, n_grp, grp * 4)
    w3 = w.reshape(n_workers, tpw * 64)
    read, upd = _gather_blend(contents, idx3, w3, xf, n_workers, grp, n_grp)

    out = _out_proj(read, W_read, min(1024, n)).reshape(b, s, h)

    upd_bf = upd.astype(jnp.bfloat16)
    new_contents = _scatter_update(contents, top_idx[:, 0], upd_bf,
                                   min(1024, m_), min(2048, n))
    return out, new_contents


# speedup vs baseline: 31.2026x; 1.1599x over previous
import functools

import jax
import jax.numpy as jnp
from jax import lax
from jax.experimental import pallas as pl
from jax.experimental.pallas import tpu as pltpu
from jax.experimental.pallas import tpu_sc as plsc

_EMA = 0.1
_EPS = 1e-08
_NEG_INF = float("-inf")
_BIG_I32 = 2 ** 30


def _anorm_body(a_ref, o_ref):
    a = a_ref[...]
    an = jnp.sqrt(jnp.sum(a * a, axis=1, keepdims=True))
    o_ref[...] = (a / jnp.maximum(an, _EPS)).astype(jnp.bfloat16)


def _addr_norm(addresses, mt):
    m_, h = addresses.shape
    return pl.pallas_call(
        _anorm_body,
        grid=(m_ // mt,),
        in_specs=[pl.BlockSpec((mt, h), lambda i: (i, 0))],
        out_specs=pl.BlockSpec((mt, h), lambda i: (i, 0)),
        out_shape=jax.ShapeDtypeStruct((m_, h), jnp.bfloat16),
    )(addresses)


def _topk_body(x_ref, wa_ref, a_ref, idx_ref, w_ref, q_scr, bv_scr, bi_scr):
    m = pl.program_id(1)
    n_m = pl.num_programs(1)
    nt, mt = q_scr.shape[0], a_ref.shape[0]

    @pl.when(m == 0)
    def _init():
        q = lax.dot_general(x_ref[...], wa_ref[...], (((1,), (1,)), ((), ())),
                            preferred_element_type=jnp.float32)
        qn = jnp.sqrt(jnp.sum(q * q, axis=1, keepdims=True))
        q_scr[...] = (q / jnp.maximum(qn, _EPS)).astype(jnp.bfloat16)

    s = lax.dot_general(a_ref[...], q_scr[...], (((1,), (1,)), ((), ())),
                        preferred_element_type=jnp.float32)

    iota_l = lax.broadcasted_iota(jnp.int32, (mt, nt), 0)
    bvs, bis = [], []
    for k in range(4):
        mx = jnp.max(s, axis=0, keepdims=True)
        ix = jnp.min(jnp.where(s == mx, iota_l, _BIG_I32), axis=0,
                     keepdims=True)
        bvs.append(mx)
        bis.append(ix)
        if k < 3:
            s = jnp.where(iota_l == ix, _NEG_INF, s)
    pad_v = jnp.full((4, nt), _NEG_INF, jnp.float32)
    pad_i = jnp.full((4, nt), _BIG_I32, jnp.int32)
    off = pl.multiple_of(m * 8, 8)
    bv_scr[pl.ds(off, 8), :] = jnp.concatenate(bvs + [pad_v], axis=0)
    bi_scr[pl.ds(off, 8), :] = jnp.concatenate(
        [b + m * mt for b in bis] + [pad_i], axis=0)

    @pl.when(m == n_m - 1)
    def _final():
        cv = bv_scr[...]
        ci = bi_scr[...]
        nvs, nis = [], []
        for _ in range(4):
            mx = jnp.max(cv, axis=0, keepdims=True)
            ix = jnp.min(jnp.where(cv == mx, ci, _BIG_I32), axis=0,
                         keepdims=True)
            nvs.append(mx)
            nis.append(ix)
            cv = jnp.where(ci == ix, _NEG_INF, cv)
        tv = jnp.concatenate(nvs, axis=0)
        e = jnp.exp(tv - jnp.max(tv, axis=0, keepdims=True))
        wsm = e / jnp.sum(e, axis=0, keepdims=True)
        w_ref[...] = jnp.broadcast_to(wsm[:, None, :],
                                      (4, 16, nt)).reshape(64, nt)
        idx_ref[...] = jnp.concatenate(nis, axis=0)


def _scores_topk(xf, w_addr, addresses, nt, mt):
    n, h = xf.shape
    m_, _ = addresses.shape
    grid = (n // nt, m_ // mt)
    return pl.pallas_call(
        _topk_body,
        grid=grid,
        in_specs=[
            pl.BlockSpec((nt, h), lambda i, j: (i, 0)),
            pl.BlockSpec((h, h), lambda i, j: (0, 0)),
            pl.BlockSpec((mt, h), lambda i, j: (j, 0)),
        ],
        out_specs=[
            pl.BlockSpec((4, nt), lambda i, j: (0, i)),
            pl.BlockSpec((64, nt), lambda i, j: (0, i)),
        ],
        out_shape=[
            jax.ShapeDtypeStruct((4, n), jnp.int32),
            jax.ShapeDtypeStruct((64, n), jnp.float32),
        ],
        scratch_shapes=[
            pltpu.VMEM((nt, h), jnp.bfloat16),
            pltpu.VMEM((8 * (m_ // mt), nt), jnp.float32),
            pltpu.VMEM((8 * (m_ // mt), nt), jnp.int32),
        ],
    )(xf, w_addr, addresses)


def _gather_blend(contents, idx3, w3, xf, n_workers, grp, n_grp):
    m_, h = contents.shape
    n, _ = xf.shape
    tpw = n // n_workers
    hc = h // 16
    mesh = plsc.VectorSubcoreMesh(core_axis_name="c", subcore_axis_name="s")

    @functools.partial(
        pl.kernel,
        out_type=[
            jax.ShapeDtypeStruct((n, h), jnp.float32),
            jax.ShapeDtypeStruct((n, h), jnp.float32),
        ],
        mesh=mesh,
        scratch_types=[
            pltpu.VMEM((n_grp, grp * 4), jnp.int32),
            pltpu.VMEM((grp * 64,), jnp.float32),
            pltpu.VMEM((grp * 4, h), jnp.float32),
            pltpu.VMEM((grp * 4, h), jnp.float32),
            pltpu.VMEM((grp, h), jnp.float32),
            pltpu.VMEM((grp, h), jnp.float32),
            pltpu.VMEM((grp, h), jnp.float32),
            pltpu.VMEM((grp, h), jnp.float32),
            pltpu.SemaphoreType.DMA,
            pltpu.SemaphoreType.DMA,
            pltpu.SemaphoreType.DMA,
            pltpu.SemaphoreType.DMA,
        ],
    )
    def k2(c_hbm, idx_hbm, w_hbm, x_hbm, read_hbm, upd_hbm,
           idx_v, w_v, rows0, rows1, x0, x1, read_v, upd_v,
           sem0, sem1, xsem0, xsem1):
        nc = lax.axis_index("c")
        ns = lax.axis_index("s")
        wid = ns * 2 + nc
        base = wid * tpw
        pltpu.sync_copy(idx_hbm.at[wid], idx_v)
        rows = [rows0, rows1]
        sems = [sem0, sem1]
        xbuf = [x0, x1]
        xsems = [xsem0, xsem1]
        for b in range(2):
            pltpu.async_copy(c_hbm.at[idx_v.at[b]], rows[b], sems[b])
            pltpu.async_copy(x_hbm.at[pl.ds(base + b * grp, grp)], xbuf[b],
                             xsems[b])

        def pair_body(i, carry):
            g0 = i * 2
            for b in range(2):
                g = g0 + b
                tok0 = base + g * grp
                pltpu.make_async_copy(c_hbm.at[idx_v.at[g]], rows[b],
                                      sems[b]).wait()
                pltpu.make_async_copy(x_hbm.at[pl.ds(tok0, grp)], xbuf[b],
                                      xsems[b]).wait()
                pltpu.sync_copy(
                    w_hbm.at[wid, pl.ds(g * grp * 64, grp * 64)], w_v)
                rv = rows[b]
                xv = xbuf[b]

                def tok_body(t, carry2):
                    wk = [w_v[pl.ds(t * 64 + k * 16, 16)] for k in range(4)]
                    for hh in range(hc):
                        sl = pl.ds(hh * 16, 16)
                        r0 = rv[t * 4 + 0, sl]
                        r1 = rv[t * 4 + 1, sl]
                        r2 = rv[t * 4 + 2, sl]
                        r3 = rv[t * 4 + 3, sl]
                        read_v[t, sl] = (wk[0] * r0 + wk[1] * r1
                                         + wk[2] * r2 + wk[3] * r3)
                        upd_v[t, sl] = _EMA * (xv[t, sl] - r0)
                    return carry2

                lax.fori_loop(0, grp, tok_body, 0)
                pltpu.sync_copy(read_v, read_hbm.at[pl.ds(tok0, grp)])
                pltpu.sync_copy(upd_v, upd_hbm.at[pl.ds(tok0, grp)])

                @pl.when(g + 2 < n_grp)
                def _prefetch():
                    pltpu.async_copy(c_hbm.at[idx_v.at[g + 2]], rows[b],
                                     sems[b])
                    pltpu.async_copy(
                        x_hbm.at[pl.ds(tok0 + 2 * grp, grp)], xbuf[b],
                        xsems[b])
            return carry

        lax.fori_loop(0, n_grp // 2, pair_body, 0)

    return k2(contents, idx3, w3, xf)


def _proj_body(r_ref, w_ref, o_ref):
    o_ref[...] = lax.dot_general(r_ref[...], w_ref[...],
                                 (((1,), (1,)), ((), ())),
                                 preferred_element_type=jnp.float32)


def _out_proj(read, w_read, nt):
    n, h = read.shape
    return pl.pallas_call(
        _proj_body,
        grid=(n // nt,),
        in_specs=[
            pl.BlockSpec((nt, h), lambda i: (i, 0)),
            pl.BlockSpec((h, h), lambda i: (0, 0)),
        ],
        out_specs=pl.BlockSpec((nt, h), lambda i: (i, 0)),
        out_shape=jax.ShapeDtypeStruct((n, h), jnp.float32),
    )(read, w_read)


def _scatter_body(c_ref, t1_ref, u_ref, o_ref):
    nn = pl.program_id(1)
    mt, nt = o_ref.shape[0], u_ref.shape[0]
    mloc = pl.program_id(0) * mt

    @pl.when(nn == 0)
    def _init():
        o_ref[...] = c_ref[...]

    idxv = t1_ref[0, 0, :]
    miota = lax.broadcasted_iota(jnp.int32, (mt, nt), 0) + mloc
    onehot = (miota == idxv[None, :]).astype(jnp.bfloat16)
    o_ref[...] += lax.dot_general(onehot, u_ref[...],
                                  (((1,), (0,)), ((), ())),
                                  preferred_element_type=jnp.float32)


def _scatter_update(contents, top1, upd_bf, mt, nt):
    m_, h = contents.shape
    n = top1.shape[0]
    t1r = top1.reshape(n // nt, 1, nt)
    grid = (m_ // mt, n // nt)
    return pl.pallas_call(
        _scatter_body,
        grid=grid,
        in_specs=[
            pl.BlockSpec((mt, h), lambda i, j: (i, 0)),
            pl.BlockSpec((1, 1, nt), lambda i, j: (j, 0, 0)),
            pl.BlockSpec((nt, h), lambda i, j: (j, 0)),
        ],
        out_specs=pl.BlockSpec((mt, h), lambda i, j: (i, 0)),
        out_shape=jax.ShapeDtypeStruct((m_, h), jnp.float32),
    )(contents, t1r, upd_bf)


def kernel(x, addresses, contents, W_addr, W_read):
    b, s, h = x.shape
    m_, _ = addresses.shape
    n = b * s
    xf = x.reshape(n, h)

    a_nb = _addr_norm(addresses, min(2048, m_))
    nt1 = min(1024, n)
    mt1 = min(1024, m_)
    ti4, w64 = _scores_topk(xf, W_addr, a_nb, nt1, mt1)
    top_idx = ti4.T
    w = w64.T

    n_workers = 32
    grp = 8
    tpw = n // n_workers
    n_grp = tpw // grp
    idx3 = top_idx.reshape(n_workers, n_grp, grp * 4)
    w3 = w.reshape(n_workers, tpw * 64)
    read, upd = _gather_blend(contents, idx3, w3, xf, n_workers, grp, n_grp)

    out = _out_proj(read, W_read, min(1024, n)).reshape(b, s, h)

    upd_bf = upd.astype(jnp.bfloat16)
    new_contents = _scatter_update(contents, top_idx[:, 0], upd_bf,
                                   min(1024, m_), min(2048, n))
    return out, new_contents


# speedup vs baseline: 35.4421x; 1.1359x over previous
import functools

import jax
import jax.numpy as jnp
from jax import lax
from jax.experimental import pallas as pl
from jax.experimental.pallas import tpu as pltpu
from jax.experimental.pallas import tpu_sc as plsc

_EMA = 0.1
_EPS = 1e-08
_NEG_INF = float("-inf")
_BIG_I32 = 2 ** 30


def _anorm_body(a_ref, o_ref):
    a = a_ref[...]
    an = jnp.sqrt(jnp.sum(a * a, axis=1, keepdims=True))
    o_ref[...] = (a / jnp.maximum(an, _EPS)).astype(jnp.bfloat16)


def _addr_norm(addresses, mt):
    m_, h = addresses.shape
    return pl.pallas_call(
        _anorm_body,
        grid=(m_ // mt,),
        in_specs=[pl.BlockSpec((mt, h), lambda i: (i, 0))],
        out_specs=pl.BlockSpec((mt, h), lambda i: (i, 0)),
        out_shape=jax.ShapeDtypeStruct((m_, h), jnp.bfloat16),
    )(addresses)


def _topk_body(x_ref, wa_ref, a_ref, idx_ref, w_ref, q_scr, bv_scr, bi_scr):
    m = pl.program_id(1)
    n_m = pl.num_programs(1)
    nt, mt = q_scr.shape[0], a_ref.shape[0]

    @pl.when(m == 0)
    def _init():
        q = lax.dot_general(x_ref[...], wa_ref[...], (((1,), (1,)), ((), ())),
                            preferred_element_type=jnp.float32)
        qn = jnp.sqrt(jnp.sum(q * q, axis=1, keepdims=True))
        q_scr[...] = (q / jnp.maximum(qn, _EPS)).astype(jnp.bfloat16)

    s = lax.dot_general(a_ref[...], q_scr[...], (((1,), (1,)), ((), ())),
                        preferred_element_type=jnp.float32)

    iota_l = lax.broadcasted_iota(jnp.int32, (mt, nt), 0)
    bvs, bis = [], []
    for k in range(4):
        mx = jnp.max(s, axis=0, keepdims=True)
        ix = jnp.min(jnp.where(s == mx, iota_l, _BIG_I32), axis=0,
                     keepdims=True)
        bvs.append(mx)
        bis.append(ix)
        if k < 3:
            s = jnp.where(iota_l == ix, _NEG_INF, s)
    pad_v = jnp.full((4, nt), _NEG_INF, jnp.float32)
    pad_i = jnp.full((4, nt), _BIG_I32, jnp.int32)
    off = pl.multiple_of(m * 8, 8)
    bv_scr[pl.ds(off, 8), :] = jnp.concatenate(bvs + [pad_v], axis=0)
    bi_scr[pl.ds(off, 8), :] = jnp.concatenate(
        [b + m * mt for b in bis] + [pad_i], axis=0)

    @pl.when(m == n_m - 1)
    def _final():
        cv = bv_scr[...]
        ci = bi_scr[...]
        nvs, nis = [], []
        for _ in range(4):
            mx = jnp.max(cv, axis=0, keepdims=True)
            ix = jnp.min(jnp.where(cv == mx, ci, _BIG_I32), axis=0,
                         keepdims=True)
            nvs.append(mx)
            nis.append(ix)
            cv = jnp.where(ci == ix, _NEG_INF, cv)
        tv = jnp.concatenate(nvs, axis=0)
        e = jnp.exp(tv - jnp.max(tv, axis=0, keepdims=True))
        wsm = e / jnp.sum(e, axis=0, keepdims=True)
        w_ref[...] = jnp.broadcast_to(wsm[:, None, :],
                                      (4, 16, nt)).reshape(64, nt)
        idx_ref[...] = jnp.concatenate(nis, axis=0)


def _scores_topk(xf, w_addr, addresses, nt, mt):
    n, h = xf.shape
    m_, _ = addresses.shape
    grid = (n // nt, m_ // mt)
    return pl.pallas_call(
        _topk_body,
        grid=grid,
        in_specs=[
            pl.BlockSpec((nt, h), lambda i, j: (i, 0)),
            pl.BlockSpec((h, h), lambda i, j: (0, 0)),
            pl.BlockSpec((mt, h), lambda i, j: (j, 0)),
        ],
        out_specs=[
            pl.BlockSpec((4, nt), lambda i, j: (0, i)),
            pl.BlockSpec((64, nt), lambda i, j: (0, i)),
        ],
        out_shape=[
            jax.ShapeDtypeStruct((4, n), jnp.int32),
            jax.ShapeDtypeStruct((64, n), jnp.float32),
        ],
        scratch_shapes=[
            pltpu.VMEM((nt, h), jnp.bfloat16),
            pltpu.VMEM((8 * (m_ // mt), nt), jnp.float32),
            pltpu.VMEM((8 * (m_ // mt), nt), jnp.int32),
        ],
    )(xf, w_addr, addresses)


def _gather_blend(contents, idx3, w3, xf, n_workers, grp, n_grp):
    m_, h = contents.shape
    n, _ = xf.shape
    tpw = n // n_workers
    hc = h // 16
    mesh = plsc.VectorSubcoreMesh(core_axis_name="c", subcore_axis_name="s")

    @functools.partial(
        pl.kernel,
        out_type=[
            jax.ShapeDtypeStruct((n, h), jnp.float32),
            jax.ShapeDtypeStruct((n, h), jnp.float32),
        ],
        mesh=mesh,
        scratch_types=[
            pltpu.VMEM((n_grp, grp * 4), jnp.int32),
            pltpu.VMEM((grp * 64,), jnp.float32),
            pltpu.VMEM((grp * 4, h), jnp.float32),
            pltpu.VMEM((grp * 4, h), jnp.float32),
            pltpu.VMEM((grp, h), jnp.float32),
            pltpu.VMEM((grp, h), jnp.float32),
            pltpu.VMEM((grp, h), jnp.float32),
            pltpu.VMEM((grp, h), jnp.float32),
            pltpu.SemaphoreType.DMA,
            pltpu.SemaphoreType.DMA,
            pltpu.SemaphoreType.DMA,
            pltpu.SemaphoreType.DMA,
        ],
    )
    def k2(c_hbm, idx_hbm, w_hbm, x_hbm, read_hbm, upd_hbm,
           idx_v, w_v, rows0, rows1, x0, x1, read_v, upd_v,
           sem0, sem1, xsem0, xsem1):
        nc = lax.axis_index("c")
        ns = lax.axis_index("s")
        wid = ns * 2 + nc
        base = wid * tpw
        pltpu.sync_copy(idx_hbm.at[wid], idx_v)
        rows = [rows0, rows1]
        sems = [sem0, sem1]
        xbuf = [x0, x1]
        xsems = [xsem0, xsem1]
        for b in range(2):
            pltpu.async_copy(c_hbm.at[idx_v.at[b]], rows[b], sems[b])
            pltpu.async_copy(x_hbm.at[pl.ds(base + b * grp, grp)], xbuf[b],
                             xsems[b])

        def pair_body(i, carry):
            g0 = i * 2
            for b in range(2):
                g = g0 + b
                tok0 = base + g * grp
                pltpu.make_async_copy(c_hbm.at[idx_v.at[g]], rows[b],
                                      sems[b]).wait()
                pltpu.make_async_copy(x_hbm.at[pl.ds(tok0, grp)], xbuf[b],
                                      xsems[b]).wait()
                pltpu.sync_copy(
                    w_hbm.at[wid, pl.ds(g * grp * 64, grp * 64)], w_v)
                rv = rows[b]
                xv = xbuf[b]

                def tok_body(t, carry2):
                    wk = [w_v[pl.ds(t * 64 + k * 16, 16)] for k in range(4)]
                    for hh in range(hc):
                        sl = pl.ds(hh * 16, 16)
                        r0 = rv[t * 4 + 0, sl]
                        r1 = rv[t * 4 + 1, sl]
                        r2 = rv[t * 4 + 2, sl]
                        r3 = rv[t * 4 + 3, sl]
                        read_v[t, sl] = (wk[0] * r0 + wk[1] * r1
                                         + wk[2] * r2 + wk[3] * r3)
                        upd_v[t, sl] = _EMA * (xv[t, sl] - r0)
                    return carry2

                lax.fori_loop(0, grp, tok_body, 0)
                pltpu.sync_copy(read_v, read_hbm.at[pl.ds(tok0, grp)])
                pltpu.sync_copy(upd_v, upd_hbm.at[pl.ds(tok0, grp)])

                @pl.when(g + 2 < n_grp)
                def _prefetch():
                    pltpu.async_copy(c_hbm.at[idx_v.at[g + 2]], rows[b],
                                     sems[b])
                    pltpu.async_copy(
                        x_hbm.at[pl.ds(tok0 + 2 * grp, grp)], xbuf[b],
                        xsems[b])
            return carry

        lax.fori_loop(0, n_grp // 2, pair_body, 0)

    return k2(contents, idx3, w3, xf)


def _proj_body(r_ref, w_ref, o_ref):
    o_ref[...] = lax.dot_general(r_ref[...], w_ref[...],
                                 (((1,), (1,)), ((), ())),
                                 preferred_element_type=jnp.float32)


def _out_proj(read, w_read, nt):
    n, h = read.shape
    return pl.pallas_call(
        _proj_body,
        grid=(n // nt,),
        in_specs=[
            pl.BlockSpec((nt, h), lambda i: (i, 0)),
            pl.BlockSpec((h, h), lambda i: (0, 0)),
        ],
        out_specs=pl.BlockSpec((nt, h), lambda i: (i, 0)),
        out_shape=jax.ShapeDtypeStruct((n, h), jnp.float32),
    )(read, w_read)


def _scatter_update_sc(contents, top1, upd, n_workers):
    m_, h = contents.shape
    n = top1.shape[0]
    tpv = n // 16
    hs = 128
    nhp = h // hs
    slots_c = m_ // 2
    rpt = slots_c // 16
    mesh = plsc.VectorSubcoreMesh(core_axis_name="c", subcore_axis_name="s")

    @functools.partial(
        pl.kernel,
        out_type=jax.ShapeDtypeStruct((m_, h), jnp.float32),
        mesh=mesh,
        scratch_types=[
            pltpu.VMEM((tpv,), jnp.int32),
            pltpu.VMEM((tpv,), jnp.int32),
            pltpu.VMEM((tpv // 2, hs), jnp.float32),
            pltpu.VMEM_SHARED((m_ // 2 + 16, hs), jnp.float32),
        ],
    )
    def k4(c_hbm, t1_hbm, u_hbm, o_hbm, idx_v, loc_v, uall, acc):
        nc = lax.axis_index("c")
        ns = lax.axis_index("s")
        base = ns * tpv
        row0 = ns * rpt
        lo = nc * slots_c
        pltpu.sync_copy(t1_hbm.at[pl.ds(base, tpv)], idx_v)

        def redir(j, carry):
            t1 = idx_v[pl.ds(j * 16, 16)]
            inb = (t1 >= lo) & (t1 < lo + slots_c)
            loc_v[pl.ds(j * 16, 16)] = jnp.where(inb, t1 - lo, slots_c)
            return carry

        lax.fori_loop(0, tpv // 16, redir, 0)

        def pass_body(hp, carry):
            hoff = hp * hs
            pltpu.sync_copy(
                c_hbm.at[pl.ds(lo + row0, rpt), pl.ds(hoff, hs)],
                acc.at[pl.ds(row0, rpt)])
            plsc.subcore_barrier()

            for half in range(2):
                toff = half * (tpv // 2)
                pltpu.sync_copy(
                    u_hbm.at[pl.ds(base + toff, tpv // 2), pl.ds(hoff, hs)],
                    uall)

                def win_body(j, carry2):
                    lv = loc_v[pl.ds(toff + j * 16, 16)]
                    pltpu.sync_copy(uall.at[pl.ds(j * 16, 16)], acc.at[lv],
                                    add=True)
                    return carry2

                lax.fori_loop(0, tpv // 32, win_body, 0)
            plsc.subcore_barrier()
            pltpu.sync_copy(acc.at[pl.ds(row0, rpt)],
                            o_hbm.at[pl.ds(lo + row0, rpt), pl.ds(hoff, hs)])
            plsc.subcore_barrier()
            return carry

        lax.fori_loop(0, nhp, pass_body, 0)

    return k4(contents, top1, upd)


def _scatter_body(c_ref, t1_ref, u_ref, o_ref):
    nn = pl.program_id(1)
    mt, nt = o_ref.shape[0], u_ref.shape[0]
    mloc = pl.program_id(0) * mt

    @pl.when(nn == 0)
    def _init():
        o_ref[...] = c_ref[...]

    idxv = t1_ref[0, 0, :]
    miota = lax.broadcasted_iota(jnp.int32, (mt, nt), 0) + mloc
    onehot = (miota == idxv[None, :]).astype(jnp.bfloat16)
    o_ref[...] += lax.dot_general(onehot, u_ref[...],
                                  (((1,), (0,)), ((), ())),
                                  preferred_element_type=jnp.float32)


def _scatter_update(contents, top1, upd_bf, mt, nt):
    m_, h = contents.shape
    n = top1.shape[0]
    t1r = top1.reshape(n // nt, 1, nt)
    grid = (m_ // mt, n // nt)
    return pl.pallas_call(
        _scatter_body,
        grid=grid,
        in_specs=[
            pl.BlockSpec((mt, h), lambda i, j: (i, 0)),
            pl.BlockSpec((1, 1, nt), lambda i, j: (j, 0, 0)),
            pl.BlockSpec((nt, h), lambda i, j: (j, 0)),
        ],
        out_specs=pl.BlockSpec((mt, h), lambda i, j: (i, 0)),
        out_shape=jax.ShapeDtypeStruct((m_, h), jnp.float32),
    )(contents, t1r, upd_bf)


def kernel(x, addresses, contents, W_addr, W_read):
    b, s, h = x.shape
    m_, _ = addresses.shape
    n = b * s
    xf = x.reshape(n, h)

    a_nb = _addr_norm(addresses, min(2048, m_))
    nt1 = min(1024, n)
    mt1 = min(1024, m_)
    ti4, w64 = _scores_topk(xf, W_addr, a_nb, nt1, mt1)
    top_idx = ti4.T
    w = w64.T

    n_workers = 32
    grp = 8
    tpw = n // n_workers
    n_grp = tpw // grp
    idx3 = top_idx.reshape(n_workers, n_grp, grp * 4)
    w3 = w.reshape(n_workers, tpw * 64)
    read, upd = _gather_blend(contents, idx3, w3, xf, n_workers, grp, n_grp)

    out = _out_proj(read, W_read, min(1024, n)).reshape(b, s, h)

    new_contents = _scatter_update_sc(contents, ti4[0], upd, n_workers)
    return out, new_contents


# speedup vs baseline: 35.8320x; 1.0110x over previous
import functools

import jax
import jax.numpy as jnp
from jax import lax
from jax.experimental import pallas as pl
from jax.experimental.pallas import tpu as pltpu
from jax.experimental.pallas import tpu_sc as plsc

_EMA = 0.1
_EPS = 1e-08
_NEG_INF = float("-inf")
_BIG_I32 = 2 ** 30


def _anorm_body(a_ref, o_ref):
    a = a_ref[...]
    an = jnp.sqrt(jnp.sum(a * a, axis=1, keepdims=True))
    o_ref[...] = (a / jnp.maximum(an, _EPS)).astype(jnp.bfloat16)


def _addr_norm(addresses, mt):
    m_, h = addresses.shape
    return pl.pallas_call(
        _anorm_body,
        grid=(m_ // mt,),
        in_specs=[pl.BlockSpec((mt, h), lambda i: (i, 0))],
        out_specs=pl.BlockSpec((mt, h), lambda i: (i, 0)),
        out_shape=jax.ShapeDtypeStruct((m_, h), jnp.bfloat16),
    )(addresses)


def _extract4(s, bm, mt, nt, bv_scr, bi_scr):
    iota_l = lax.broadcasted_iota(jnp.int32, (mt, nt), 0)
    bvs, bis = [], []
    for k in range(4):
        mx = jnp.max(s, axis=0, keepdims=True)
        ix = jnp.min(jnp.where(s == mx, iota_l, _BIG_I32), axis=0,
                     keepdims=True)
        bvs.append(mx)
        bis.append(ix)
        if k < 3:
            s = jnp.where(iota_l == ix, _NEG_INF, s)
    pad_v = jnp.full((4, nt), _NEG_INF, jnp.float32)
    pad_i = jnp.full((4, nt), _BIG_I32, jnp.int32)
    off = pl.multiple_of(bm * 8 + 8, 8)
    bv_scr[pl.ds(off, 8), :] = jnp.concatenate(bvs + [pad_v], axis=0)
    bi_scr[pl.ds(off, 8), :] = jnp.concatenate(
        [b + bm * mt for b in bis] + [pad_i], axis=0)


def _topk_body(x_ref, wa_ref, a_ref, idx_ref, w_ref, q_scr, s_scr,
               bv_scr, bi_scr):
    m = pl.program_id(1)
    n_m = pl.num_programs(1)
    nt, mt = q_scr.shape[0], a_ref.shape[0]

    @pl.when(m == 0)
    def _init():
        q = lax.dot_general(x_ref[...], wa_ref[...], (((1,), (1,)), ((), ())),
                            preferred_element_type=jnp.float32)
        qn = jnp.sqrt(jnp.sum(q * q, axis=1, keepdims=True))
        q_scr[...] = (q / jnp.maximum(qn, _EPS)).astype(jnp.bfloat16)

    s = lax.dot_general(a_ref[...], q_scr[...], (((1,), (1,)), ((), ())),
                        preferred_element_type=jnp.float32)

    sprev = s_scr[...]
    _extract4(sprev, m - 1, mt, nt, bv_scr, bi_scr)
    s_scr[...] = s

    @pl.when(m == n_m - 1)
    def _final():
        _extract4(s, n_m - 1, mt, nt, bv_scr, bi_scr)
        cv = bv_scr[pl.ds(8, 8 * n_m), :]
        ci = bi_scr[pl.ds(8, 8 * n_m), :]
        nvs, nis = [], []
        for _ in range(4):
            mx = jnp.max(cv, axis=0, keepdims=True)
            ix = jnp.min(jnp.where(cv == mx, ci, _BIG_I32), axis=0,
                         keepdims=True)
            nvs.append(mx)
            nis.append(ix)
            cv = jnp.where(ci == ix, _NEG_INF, cv)
        tv = jnp.concatenate(nvs, axis=0)
        e = jnp.exp(tv - jnp.max(tv, axis=0, keepdims=True))
        wsm = e / jnp.sum(e, axis=0, keepdims=True)
        w_ref[...] = jnp.broadcast_to(wsm[:, None, :],
                                      (4, 16, nt)).reshape(64, nt)
        idx_ref[...] = jnp.concatenate(nis, axis=0)


def _scores_topk(xf, w_addr, addresses, nt, mt):
    n, h = xf.shape
    m_, _ = addresses.shape
    grid = (n // nt, m_ // mt)
    return pl.pallas_call(
        _topk_body,
        grid=grid,
        in_specs=[
            pl.BlockSpec((nt, h), lambda i, j: (i, 0)),
            pl.BlockSpec((h, h), lambda i, j: (0, 0)),
            pl.BlockSpec((mt, h), lambda i, j: (j, 0)),
        ],
        out_specs=[
            pl.BlockSpec((4, nt), lambda i, j: (0, i)),
            pl.BlockSpec((64, nt), lambda i, j: (0, i)),
        ],
        out_shape=[
            jax.ShapeDtypeStruct((4, n), jnp.int32),
            jax.ShapeDtypeStruct((64, n), jnp.float32),
        ],
        scratch_shapes=[
            pltpu.VMEM((nt, h), jnp.bfloat16),
            pltpu.VMEM((mt, nt), jnp.float32),
            pltpu.VMEM((8 * (m_ // mt) + 8, nt), jnp.float32),
            pltpu.VMEM((8 * (m_ // mt) + 8, nt), jnp.int32),
        ],
    )(xf, w_addr, addresses)


def _gather_blend(contents, idx3, w3, xf, n_workers, grp, n_grp):
    m_, h = contents.shape
    n, _ = xf.shape
    tpw = n // n_workers
    hc = h // 16
    mesh = plsc.VectorSubcoreMesh(core_axis_name="c", subcore_axis_name="s")

    @functools.partial(
        pl.kernel,
        out_type=[
            jax.ShapeDtypeStruct((n, h), jnp.float32),
            jax.ShapeDtypeStruct((n, h), jnp.float32),
        ],
        mesh=mesh,
        scratch_types=[
            pltpu.VMEM((n_grp, grp * 4), jnp.int32),
            pltpu.VMEM((grp * 64,), jnp.float32),
            pltpu.VMEM((grp * 4, h), jnp.float32),
            pltpu.VMEM((grp * 4, h), jnp.float32),
            pltpu.VMEM((grp, h), jnp.float32),
            pltpu.VMEM((grp, h), jnp.float32),
            pltpu.VMEM((grp, h), jnp.float32),
            pltpu.VMEM((grp, h), jnp.float32),
            pltpu.SemaphoreType.DMA,
            pltpu.SemaphoreType.DMA,
            pltpu.SemaphoreType.DMA,
            pltpu.SemaphoreType.DMA,
        ],
    )
    def k2(c_hbm, idx_hbm, w_hbm, x_hbm, read_hbm, upd_hbm,
           idx_v, w_v, rows0, rows1, x0, x1, read_v, upd_v,
           sem0, sem1, xsem0, xsem1):
        nc = lax.axis_index("c")
        ns = lax.axis_index("s")
        wid = ns * 2 + nc
        base = wid * tpw
        pltpu.sync_copy(idx_hbm.at[wid], idx_v)
        rows = [rows0, rows1]
        sems = [sem0, sem1]
        xbuf = [x0, x1]
        xsems = [xsem0, xsem1]
        for b in range(2):
            pltpu.async_copy(c_hbm.at[idx_v.at[b]], rows[b], sems[b])
            pltpu.async_copy(x_hbm.at[pl.ds(base + b * grp, grp)], xbuf[b],
                             xsems[b])

        def pair_body(i, carry):
            g0 = i * 2
            for b in range(2):
                g = g0 + b
                tok0 = base + g * grp
                pltpu.make_async_copy(c_hbm.at[idx_v.at[g]], rows[b],
                                      sems[b]).wait()
                pltpu.make_async_copy(x_hbm.at[pl.ds(tok0, grp)], xbuf[b],
                                      xsems[b]).wait()
                pltpu.sync_copy(
                    w_hbm.at[wid, pl.ds(g * grp * 64, grp * 64)], w_v)
                rv = rows[b]
                xv = xbuf[b]

                def tok_body(t, carry2):
                    wk = [w_v[pl.ds(t * 64 + k * 16, 16)] for k in range(4)]
                    for hh in range(hc):
                        sl = pl.ds(hh * 16, 16)
                        r0 = rv[t * 4 + 0, sl]
                        r1 = rv[t * 4 + 1, sl]
                        r2 = rv[t * 4 + 2, sl]
                        r3 = rv[t * 4 + 3, sl]
                        read_v[t, sl] = (wk[0] * r0 + wk[1] * r1
                                         + wk[2] * r2 + wk[3] * r3)
                        upd_v[t, sl] = _EMA * (xv[t, sl] - r0)
                    return carry2

                lax.fori_loop(0, grp, tok_body, 0)
                pltpu.sync_copy(read_v, read_hbm.at[pl.ds(tok0, grp)])
                pltpu.sync_copy(upd_v, upd_hbm.at[pl.ds(tok0, grp)])

                @pl.when(g + 2 < n_grp)
                def _prefetch():
                    pltpu.async_copy(c_hbm.at[idx_v.at[g + 2]], rows[b],
                                     sems[b])
                    pltpu.async_copy(
                        x_hbm.at[pl.ds(tok0 + 2 * grp, grp)], xbuf[b],
                        xsems[b])
            return carry

        lax.fori_loop(0, n_grp // 2, pair_body, 0)

    return k2(contents, idx3, w3, xf)


def _proj_body(r_ref, w_ref, o_ref):
    o_ref[...] = lax.dot_general(r_ref[...], w_ref[...],
                                 (((1,), (1,)), ((), ())),
                                 preferred_element_type=jnp.float32)


def _out_proj(read, w_read, nt):
    n, h = read.shape
    return pl.pallas_call(
        _proj_body,
        grid=(n // nt,),
        in_specs=[
            pl.BlockSpec((nt, h), lambda i: (i, 0)),
            pl.BlockSpec((h, h), lambda i: (0, 0)),
        ],
        out_specs=pl.BlockSpec((nt, h), lambda i: (i, 0)),
        out_shape=jax.ShapeDtypeStruct((n, h), jnp.float32),
    )(read, w_read)


def _scatter_update_sc(contents, top1, upd, n_workers):
    m_, h = contents.shape
    n = top1.shape[0]
    tpv = n // 16
    hs = 128
    nhp = h // hs
    slots_c = m_ // 2
    rpt = slots_c // 16
    mesh = plsc.VectorSubcoreMesh(core_axis_name="c", subcore_axis_name="s")

    @functools.partial(
        pl.kernel,
        out_type=jax.ShapeDtypeStruct((m_, h), jnp.float32),
        mesh=mesh,
        scratch_types=[
            pltpu.VMEM((tpv,), jnp.int32),
            pltpu.VMEM((tpv,), jnp.int32),
            pltpu.VMEM((tpv // 2, hs), jnp.float32),
            pltpu.VMEM_SHARED((m_ // 2 + 16, hs), jnp.float32),
        ],
    )
    def k4(c_hbm, t1_hbm, u_hbm, o_hbm, idx_v, loc_v, uall, acc):
        nc = lax.axis_index("c")
        ns = lax.axis_index("s")
        base = ns * tpv
        row0 = ns * rpt
        lo = nc * slots_c
        pltpu.sync_copy(t1_hbm.at[pl.ds(base, tpv)], idx_v)

        def redir(j, carry):
            t1 = idx_v[pl.ds(j * 16, 16)]
            inb = (t1 >= lo) & (t1 < lo + slots_c)
            loc_v[pl.ds(j * 16, 16)] = jnp.where(inb, t1 - lo, slots_c)
            return carry

        lax.fori_loop(0, tpv // 16, redir, 0)

        def pass_body(hp, carry):
            hoff = hp * hs
            pltpu.sync_copy(
                c_hbm.at[pl.ds(lo + row0, rpt), pl.ds(hoff, hs)],
                acc.at[pl.ds(row0, rpt)])
            plsc.subcore_barrier()

            for half in range(2):
                toff = half * (tpv // 2)
                pltpu.sync_copy(
                    u_hbm.at[pl.ds(base + toff, tpv // 2), pl.ds(hoff, hs)],
                    uall)

                def win_body(j, carry2):
                    lv = loc_v[pl.ds(toff + j * 16, 16)]
                    pltpu.sync_copy(uall.at[pl.ds(j * 16, 16)], acc.at[lv],
                                    add=True)
                    return carry2

                lax.fori_loop(0, tpv // 32, win_body, 0)
            plsc.subcore_barrier()
            pltpu.sync_copy(acc.at[pl.ds(row0, rpt)],
                            o_hbm.at[pl.ds(lo + row0, rpt), pl.ds(hoff, hs)])
            plsc.subcore_barrier()
            return carry

        lax.fori_loop(0, nhp, pass_body, 0)

    return k4(contents, top1, upd)


def _scatter_body(c_ref, t1_ref, u_ref, o_ref):
    nn = pl.program_id(1)
    mt, nt = o_ref.shape[0], u_ref.shape[0]
    mloc = pl.program_id(0) * mt

    @pl.when(nn == 0)
    def _init():
        o_ref[...] = c_ref[...]

    idxv = t1_ref[0, 0, :]
    miota = lax.broadcasted_iota(jnp.int32, (mt, nt), 0) + mloc
    onehot = (miota == idxv[None, :]).astype(jnp.bfloat16)
    o_ref[...] += lax.dot_general(onehot, u_ref[...],
                                  (((1,), (0,)), ((), ())),
                                  preferred_element_type=jnp.float32)


def _scatter_update(contents, top1, upd_bf, mt, nt):
    m_, h = contents.shape
    n = top1.shape[0]
    t1r = top1.reshape(n // nt, 1, nt)
    grid = (m_ // mt, n // nt)
    return pl.pallas_call(
        _scatter_body,
        grid=grid,
        in_specs=[
            pl.BlockSpec((mt, h), lambda i, j: (i, 0)),
            pl.BlockSpec((1, 1, nt), lambda i, j: (j, 0, 0)),
            pl.BlockSpec((nt, h), lambda i, j: (j, 0)),
        ],
        out_specs=pl.BlockSpec((mt, h), lambda i, j: (i, 0)),
        out_shape=jax.ShapeDtypeStruct((m_, h), jnp.float32),
    )(contents, t1r, upd_bf)


def kernel(x, addresses, contents, W_addr, W_read):
    b, s, h = x.shape
    m_, _ = addresses.shape
    n = b * s
    xf = x.reshape(n, h)

    a_nb = _addr_norm(addresses, min(2048, m_))
    nt1 = min(1024, n)
    mt1 = min(1024, m_)
    ti4, w64 = _scores_topk(xf, W_addr, a_nb, nt1, mt1)
    top_idx = ti4.T
    w = w64.T

    n_workers = 32
    grp = 8
    tpw = n // n_workers
    n_grp = tpw // grp
    idx3 = top_idx.reshape(n_workers, n_grp, grp * 4)
    w3 = w.reshape(n_workers, tpw * 64)
    read, upd = _gather_blend(contents, idx3, w3, xf, n_workers, grp, n_grp)

    out = _out_proj(read, W_read, min(1024, n)).reshape(b, s, h)

    new_contents = _scatter_update_sc(contents, ti4[0], upd, n_workers)
    return out, new_contents


# speedup vs baseline: 35.8325x; 1.0000x over previous
import functools

import jax
import jax.numpy as jnp
from jax import lax
from jax.experimental import pallas as pl
from jax.experimental.pallas import tpu as pltpu
from jax.experimental.pallas import tpu_sc as plsc

_EMA = 0.1
_EPS = 1e-08
_NEG_INF = float("-inf")
_BIG_I32 = 2 ** 30


def _anorm_body(a_ref, o_ref):
    a = a_ref[...]
    an = jnp.sqrt(jnp.sum(a * a, axis=1, keepdims=True))
    o_ref[...] = (a / jnp.maximum(an, _EPS)).astype(jnp.bfloat16)


def _addr_norm(addresses, mt):
    m_, h = addresses.shape
    return pl.pallas_call(
        _anorm_body,
        grid=(m_ // mt,),
        in_specs=[pl.BlockSpec((mt, h), lambda i: (i, 0))],
        out_specs=pl.BlockSpec((mt, h), lambda i: (i, 0)),
        out_shape=jax.ShapeDtypeStruct((m_, h), jnp.bfloat16),
    )(addresses)


def _extract4(s, bm, mt, nt, bv_scr, bi_scr):
    iota_l = lax.broadcasted_iota(jnp.int32, (mt, nt), 0)
    bvs, bis = [], []
    for k in range(4):
        mx = jnp.max(s, axis=0, keepdims=True)
        ix = jnp.min(jnp.where(s == mx, iota_l, _BIG_I32), axis=0,
                     keepdims=True)
        bvs.append(mx)
        bis.append(ix)
        if k < 3:
            s = jnp.where(iota_l == ix, _NEG_INF, s)
    pad_v = jnp.full((4, nt), _NEG_INF, jnp.float32)
    pad_i = jnp.full((4, nt), _BIG_I32, jnp.int32)
    off = pl.multiple_of(bm * 8 + 8, 8)
    bv_scr[pl.ds(off, 8), :] = jnp.concatenate(bvs + [pad_v], axis=0)
    bi_scr[pl.ds(off, 8), :] = jnp.concatenate(
        [b + bm * mt for b in bis] + [pad_i], axis=0)


def _qnorm_body(x_ref, wa_ref, o_ref):
    q = lax.dot_general(x_ref[...], wa_ref[...], (((1,), (1,)), ((), ())),
                        preferred_element_type=jnp.float32)
    qn = jnp.sqrt(jnp.sum(q * q, axis=1, keepdims=True))
    o_ref[...] = (q / jnp.maximum(qn, _EPS)).astype(jnp.bfloat16)


def _q_norm(xf, w_addr, nt):
    n, h = xf.shape
    return pl.pallas_call(
        _qnorm_body,
        grid=(n // nt,),
        in_specs=[
            pl.BlockSpec((nt, h), lambda i: (i, 0)),
            pl.BlockSpec((h, h), lambda i: (0, 0)),
        ],
        out_specs=pl.BlockSpec((nt, h), lambda i: (i, 0)),
        out_shape=jax.ShapeDtypeStruct((n, h), jnp.bfloat16),
    )(xf, w_addr)


def _topk_body(qb_ref, a_ref, idx_ref, w_ref, s_scr, bv_scr, bi_scr):
    m = pl.program_id(1)
    n_m = pl.num_programs(1)
    nt, mt = qb_ref.shape[0], a_ref.shape[0]

    s = lax.dot_general(a_ref[...], qb_ref[...], (((1,), (1,)), ((), ())),
                        preferred_element_type=jnp.float32)

    sprev = s_scr[...]
    _extract4(sprev, m - 1, mt, nt, bv_scr, bi_scr)
    s_scr[...] = s

    @pl.when(m == n_m - 1)
    def _final():
        _extract4(s, n_m - 1, mt, nt, bv_scr, bi_scr)
        cv = bv_scr[pl.ds(8, 8 * n_m), :]
        ci = bi_scr[pl.ds(8, 8 * n_m), :]
        nvs, nis = [], []
        for _ in range(4):
            mx = jnp.max(cv, axis=0, keepdims=True)
            ix = jnp.min(jnp.where(cv == mx, ci, _BIG_I32), axis=0,
                         keepdims=True)
            nvs.append(mx)
            nis.append(ix)
            cv = jnp.where(ci == ix, _NEG_INF, cv)
        tv = jnp.concatenate(nvs, axis=0)
        e = jnp.exp(tv - jnp.max(tv, axis=0, keepdims=True))
        wsm = e / jnp.sum(e, axis=0, keepdims=True)
        w_ref[...] = jnp.broadcast_to(wsm[:, None, :],
                                      (4, 16, nt)).reshape(64, nt)
        idx_ref[...] = jnp.concatenate(nis, axis=0)


def _scores_topk(qb, addresses, nt, mt):
    n, h = qb.shape
    m_, _ = addresses.shape
    grid = (n // nt, m_ // mt)
    return pl.pallas_call(
        _topk_body,
        grid=grid,
        in_specs=[
            pl.BlockSpec((nt, h), lambda i, j: (i, 0)),
            pl.BlockSpec((mt, h), lambda i, j: (j, 0)),
        ],
        out_specs=[
            pl.BlockSpec((4, nt), lambda i, j: (0, i)),
            pl.BlockSpec((64, nt), lambda i, j: (0, i)),
        ],
        out_shape=[
            jax.ShapeDtypeStruct((4, n), jnp.int32),
            jax.ShapeDtypeStruct((64, n), jnp.float32),
        ],
        scratch_shapes=[
            pltpu.VMEM((mt, nt), jnp.float32),
            pltpu.VMEM((8 * (m_ // mt) + 8, nt), jnp.float32),
            pltpu.VMEM((8 * (m_ // mt) + 8, nt), jnp.int32),
        ],
    )(qb, addresses)


def _gather_blend(contents, idx3, w3, xf, n_workers, grp, n_grp):
    m_, h = contents.shape
    n, _ = xf.shape
    tpw = n // n_workers
    hc = h // 16
    mesh = plsc.VectorSubcoreMesh(core_axis_name="c", subcore_axis_name="s")

    @functools.partial(
        pl.kernel,
        out_type=[
            jax.ShapeDtypeStruct((n, h), jnp.float32),
            jax.ShapeDtypeStruct((n, h), jnp.float32),
        ],
        mesh=mesh,
        scratch_types=[
            pltpu.VMEM((n_grp, grp * 4), jnp.int32),
            pltpu.VMEM((grp * 64,), jnp.float32),
            pltpu.VMEM((grp * 4, h), jnp.float32),
            pltpu.VMEM((grp * 4, h), jnp.float32),
            pltpu.VMEM((grp, h), jnp.float32),
            pltpu.VMEM((grp, h), jnp.float32),
            pltpu.VMEM((grp, h), jnp.float32),
            pltpu.VMEM((grp, h), jnp.float32),
            pltpu.SemaphoreType.DMA,
            pltpu.SemaphoreType.DMA,
            pltpu.SemaphoreType.DMA,
            pltpu.SemaphoreType.DMA,
        ],
    )
    def k2(c_hbm, idx_hbm, w_hbm, x_hbm, read_hbm, upd_hbm,
           idx_v, w_v, rows0, rows1, x0, x1, read_v, upd_v,
           sem0, sem1, xsem0, xsem1):
        nc = lax.axis_index("c")
        ns = lax.axis_index("s")
        wid = ns * 2 + nc
        base = wid * tpw
        pltpu.sync_copy(idx_hbm.at[wid], idx_v)
        rows = [rows0, rows1]
        sems = [sem0, sem1]
        xbuf = [x0, x1]
        xsems = [xsem0, xsem1]
        for b in range(2):
            pltpu.async_copy(c_hbm.at[idx_v.at[b]], rows[b], sems[b])
            pltpu.async_copy(x_hbm.at[pl.ds(base + b * grp, grp)], xbuf[b],
                             xsems[b])

        def pair_body(i, carry):
            g0 = i * 2
            for b in range(2):
                g = g0 + b
                tok0 = base + g * grp
                pltpu.make_async_copy(c_hbm.at[idx_v.at[g]], rows[b],
                                      sems[b]).wait()
                pltpu.make_async_copy(x_hbm.at[pl.ds(tok0, grp)], xbuf[b],
                                      xsems[b]).wait()
                pltpu.sync_copy(
                    w_hbm.at[wid, pl.ds(g * grp * 64, grp * 64)], w_v)
                rv = rows[b]
                xv = xbuf[b]

                def tok_body(t, carry2):
                    wk = [w_v[pl.ds(t * 64 + k * 16, 16)] for k in range(4)]
                    for hh in range(hc):
                        sl = pl.ds(hh * 16, 16)
                        r0 = rv[t * 4 + 0, sl]
                        r1 = rv[t * 4 + 1, sl]
                        r2 = rv[t * 4 + 2, sl]
                        r3 = rv[t * 4 + 3, sl]
                        read_v[t, sl] = (wk[0] * r0 + wk[1] * r1
                                         + wk[2] * r2 + wk[3] * r3)
                        upd_v[t, sl] = _EMA * (xv[t, sl] - r0)
                    return carry2

                lax.fori_loop(0, grp, tok_body, 0)
                pltpu.sync_copy(read_v, read_hbm.at[pl.ds(tok0, grp)])
                pltpu.sync_copy(upd_v, upd_hbm.at[pl.ds(tok0, grp)])

                @pl.when(g + 2 < n_grp)
                def _prefetch():
                    pltpu.async_copy(c_hbm.at[idx_v.at[g + 2]], rows[b],
                                     sems[b])
                    pltpu.async_copy(
                        x_hbm.at[pl.ds(tok0 + 2 * grp, grp)], xbuf[b],
                        xsems[b])
            return carry

        lax.fori_loop(0, n_grp // 2, pair_body, 0)

    return k2(contents, idx3, w3, xf)


def _proj_body(r_ref, w_ref, o_ref):
    o_ref[...] = lax.dot_general(r_ref[...], w_ref[...],
                                 (((1,), (1,)), ((), ())),
                                 preferred_element_type=jnp.float32)


def _out_proj(read, w_read, nt):
    n, h = read.shape
    return pl.pallas_call(
        _proj_body,
        grid=(n // nt,),
        in_specs=[
            pl.BlockSpec((nt, h), lambda i: (i, 0)),
            pl.BlockSpec((h, h), lambda i: (0, 0)),
        ],
        out_specs=pl.BlockSpec((nt, h), lambda i: (i, 0)),
        out_shape=jax.ShapeDtypeStruct((n, h), jnp.float32),
    )(read, w_read)


def _scatter_update_sc(contents, top1, upd, n_workers):
    m_, h = contents.shape
    n = top1.shape[0]
    tpv = n // 16
    hs = 128
    nhp = h // hs
    slots_c = m_ // 2
    rpt = slots_c // 16
    mesh = plsc.VectorSubcoreMesh(core_axis_name="c", subcore_axis_name="s")

    @functools.partial(
        pl.kernel,
        out_type=jax.ShapeDtypeStruct((m_, h), jnp.float32),
        mesh=mesh,
        scratch_types=[
            pltpu.VMEM((tpv,), jnp.int32),
            pltpu.VMEM((tpv,), jnp.int32),
            pltpu.VMEM((tpv // 2, hs), jnp.float32),
            pltpu.VMEM_SHARED((m_ // 2 + 16, hs), jnp.float32),
        ],
    )
    def k4(c_hbm, t1_hbm, u_hbm, o_hbm, idx_v, loc_v, uall, acc):
        nc = lax.axis_index("c")
        ns = lax.axis_index("s")
        base = ns * tpv
        row0 = ns * rpt
        lo = nc * slots_c
        pltpu.sync_copy(t1_hbm.at[pl.ds(base, tpv)], idx_v)

        def redir(j, carry):
            t1 = idx_v[pl.ds(j * 16, 16)]
            inb = (t1 >= lo) & (t1 < lo + slots_c)
            loc_v[pl.ds(j * 16, 16)] = jnp.where(inb, t1 - lo, slots_c)
            return carry

        lax.fori_loop(0, tpv // 16, redir, 0)

        def pass_body(hp, carry):
            hoff = hp * hs
            pltpu.sync_copy(
                c_hbm.at[pl.ds(lo + row0, rpt), pl.ds(hoff, hs)],
                acc.at[pl.ds(row0, rpt)])
            plsc.subcore_barrier()

            for half in range(2):
                toff = half * (tpv // 2)
                pltpu.sync_copy(
                    u_hbm.at[pl.ds(base + toff, tpv // 2), pl.ds(hoff, hs)],
                    uall)

                def win_body(j, carry2):
                    lv = loc_v[pl.ds(toff + j * 16, 16)]
                    pltpu.sync_copy(uall.at[pl.ds(j * 16, 16)], acc.at[lv],
                                    add=True)
                    return carry2

                lax.fori_loop(0, tpv // 32, win_body, 0)
            plsc.subcore_barrier()
            pltpu.sync_copy(acc.at[pl.ds(row0, rpt)],
                            o_hbm.at[pl.ds(lo + row0, rpt), pl.ds(hoff, hs)])
            plsc.subcore_barrier()
            return carry

        lax.fori_loop(0, nhp, pass_body, 0)

    return k4(contents, top1, upd)


def _scatter_body(c_ref, t1_ref, u_ref, o_ref):
    nn = pl.program_id(1)
    mt, nt = o_ref.shape[0], u_ref.shape[0]
    mloc = pl.program_id(0) * mt

    @pl.when(nn == 0)
    def _init():
        o_ref[...] = c_ref[...]

    idxv = t1_ref[0, 0, :]
    miota = lax.broadcasted_iota(jnp.int32, (mt, nt), 0) + mloc
    onehot = (miota == idxv[None, :]).astype(jnp.bfloat16)
    o_ref[...] += lax.dot_general(onehot, u_ref[...],
                                  (((1,), (0,)), ((), ())),
                                  preferred_element_type=jnp.float32)


def _scatter_update(contents, top1, upd_bf, mt, nt):
    m_, h = contents.shape
    n = top1.shape[0]
    t1r = top1.reshape(n // nt, 1, nt)
    grid = (m_ // mt, n // nt)
    return pl.pallas_call(
        _scatter_body,
        grid=grid,
        in_specs=[
            pl.BlockSpec((mt, h), lambda i, j: (i, 0)),
            pl.BlockSpec((1, 1, nt), lambda i, j: (j, 0, 0)),
            pl.BlockSpec((nt, h), lambda i, j: (j, 0)),
        ],
        out_specs=pl.BlockSpec((mt, h), lambda i, j: (i, 0)),
        out_shape=jax.ShapeDtypeStruct((m_, h), jnp.float32),
    )(contents, t1r, upd_bf)


def kernel(x, addresses, contents, W_addr, W_read):
    b, s, h = x.shape
    m_, _ = addresses.shape
    n = b * s
    xf = x.reshape(n, h)

    a_nb = _addr_norm(addresses, min(2048, m_))
    qb = _q_norm(xf, W_addr, min(1024, n))
    nt1 = min(2048, n)
    mt1 = min(1024, m_)
    ti4, w64 = _scores_topk(qb, a_nb, nt1, mt1)
    top_idx = ti4.T
    w = w64.T

    n_workers = 32
    grp = 8
    tpw = n // n_workers
    n_grp = tpw // grp
    idx3 = top_idx.reshape(n_workers, n_grp, grp * 4)
    w3 = w.reshape(n_workers, tpw * 64)
    read, upd = _gather_blend(contents, idx3, w3, xf, n_workers, grp, n_grp)

    out = _out_proj(read, W_read, min(1024, n)).reshape(b, s, h)

    new_contents = _scatter_update_sc(contents, ti4[0], upd, n_workers)
    return out, new_contents


# speedup vs baseline: 35.9020x; 1.0019x over previous
import functools

import jax
import jax.numpy as jnp
from jax import lax
from jax.experimental import pallas as pl
from jax.experimental.pallas import tpu as pltpu
from jax.experimental.pallas import tpu_sc as plsc

_EMA = 0.1
_EPS = 1e-08
_NEG_INF = float("-inf")
_BIG_I32 = 2 ** 30


def _anorm_body(a_ref, o_ref):
    a = a_ref[...]
    an = jnp.sqrt(jnp.sum(a * a, axis=1, keepdims=True))
    o_ref[...] = (a / jnp.maximum(an, _EPS)).astype(jnp.bfloat16)


def _addr_norm(addresses, mt):
    m_, h = addresses.shape
    return pl.pallas_call(
        _anorm_body,
        grid=(m_ // mt,),
        in_specs=[pl.BlockSpec((mt, h), lambda i: (i, 0))],
        out_specs=pl.BlockSpec((mt, h), lambda i: (i, 0)),
        out_shape=jax.ShapeDtypeStruct((m_, h), jnp.bfloat16),
    )(addresses)


def _extract4(s, bm, mt, nt, bv_scr, bi_scr):
    iota_l = lax.broadcasted_iota(jnp.int32, (mt, nt), 0)
    bvs, bis = [], []
    for k in range(4):
        mx = jnp.max(s, axis=0, keepdims=True)
        ix = jnp.min(jnp.where(s == mx, iota_l, _BIG_I32), axis=0,
                     keepdims=True)
        bvs.append(mx)
        bis.append(ix)
        if k < 3:
            s = jnp.where(iota_l == ix, _NEG_INF, s)
    pad_v = jnp.full((4, nt), _NEG_INF, jnp.float32)
    pad_i = jnp.full((4, nt), _BIG_I32, jnp.int32)
    off = pl.multiple_of(bm * 8 + 8, 8)
    bv_scr[pl.ds(off, 8), :] = jnp.concatenate(bvs + [pad_v], axis=0)
    bi_scr[pl.ds(off, 8), :] = jnp.concatenate(
        [b + bm * mt for b in bis] + [pad_i], axis=0)


def _qnorm_body(x_ref, wa_ref, o_ref):
    q = lax.dot_general(x_ref[...], wa_ref[...], (((1,), (1,)), ((), ())),
                        preferred_element_type=jnp.float32)
    qn = jnp.sqrt(jnp.sum(q * q, axis=1, keepdims=True))
    o_ref[...] = (q / jnp.maximum(qn, _EPS)).astype(jnp.bfloat16)


def _q_norm(xf, w_addr, nt):
    n, h = xf.shape
    return pl.pallas_call(
        _qnorm_body,
        grid=(n // nt,),
        in_specs=[
            pl.BlockSpec((nt, h), lambda i: (i, 0)),
            pl.BlockSpec((h, h), lambda i: (0, 0)),
        ],
        out_specs=pl.BlockSpec((nt, h), lambda i: (i, 0)),
        out_shape=jax.ShapeDtypeStruct((n, h), jnp.bfloat16),
    )(xf, w_addr)


def _topk_body(qb_ref, a_ref, idx_ref, w_ref, s_scr, bv_scr, bi_scr):
    m = pl.program_id(1)
    n_m = pl.num_programs(1)
    nt, mt = qb_ref.shape[0], a_ref.shape[0]

    s = lax.dot_general(a_ref[...], qb_ref[...], (((1,), (1,)), ((), ())),
                        preferred_element_type=jnp.float32)

    sprev = s_scr[...]
    _extract4(sprev, m - 1, mt, nt, bv_scr, bi_scr)
    s_scr[...] = s

    @pl.when(m == n_m - 1)
    def _final():
        _extract4(s, n_m - 1, mt, nt, bv_scr, bi_scr)
        cv = bv_scr[pl.ds(8, 8 * n_m), :]
        ci = bi_scr[pl.ds(8, 8 * n_m), :]
        nvs, nis = [], []
        for _ in range(4):
            mx = jnp.max(cv, axis=0, keepdims=True)
            ix = jnp.min(jnp.where(cv == mx, ci, _BIG_I32), axis=0,
                         keepdims=True)
            nvs.append(mx)
            nis.append(ix)
            cv = jnp.where(ci == ix, _NEG_INF, cv)
        tv = jnp.concatenate(nvs, axis=0)
        e = jnp.exp(tv - jnp.max(tv, axis=0, keepdims=True))
        wsm = e / jnp.sum(e, axis=0, keepdims=True)
        w_ref[...] = jnp.broadcast_to(wsm[:, None, :],
                                      (4, 16, nt)).reshape(64, nt)
        idx_ref[...] = jnp.concatenate(nis, axis=0)


def _scores_topk(qb, addresses, nt, mt):
    n, h = qb.shape
    m_, _ = addresses.shape
    grid = (n // nt, m_ // mt)
    return pl.pallas_call(
        _topk_body,
        grid=grid,
        in_specs=[
            pl.BlockSpec((nt, h), lambda i, j: (i, 0)),
            pl.BlockSpec((mt, h), lambda i, j: (j, 0)),
        ],
        out_specs=[
            pl.BlockSpec((4, nt), lambda i, j: (0, i)),
            pl.BlockSpec((64, nt), lambda i, j: (0, i)),
        ],
        out_shape=[
            jax.ShapeDtypeStruct((4, n), jnp.int32),
            jax.ShapeDtypeStruct((64, n), jnp.float32),
        ],
        scratch_shapes=[
            pltpu.VMEM((mt, nt), jnp.float32),
            pltpu.VMEM((8 * (m_ // mt) + 8, nt), jnp.float32),
            pltpu.VMEM((8 * (m_ // mt) + 8, nt), jnp.int32),
        ],
    )(qb, addresses)


def _gather_blend(contents, idx3, w3, xf, n_workers, grp, n_grp):
    m_, h = contents.shape
    n, _ = xf.shape
    tpw = n // n_workers
    hc = h // 16
    mesh = plsc.VectorSubcoreMesh(core_axis_name="c", subcore_axis_name="s")

    @functools.partial(
        pl.kernel,
        out_type=[
            jax.ShapeDtypeStruct((n, h), jnp.float32),
            jax.ShapeDtypeStruct((n, h), jnp.float32),
        ],
        mesh=mesh,
        scratch_types=[
            pltpu.VMEM((n_grp, grp * 4), jnp.int32),
            pltpu.VMEM((grp * 64,), jnp.float32),
            pltpu.VMEM((grp * 4, h), jnp.float32),
            pltpu.VMEM((grp * 4, h), jnp.float32),
            pltpu.VMEM((grp, h), jnp.float32),
            pltpu.VMEM((grp, h), jnp.float32),
            pltpu.VMEM((grp, h), jnp.float32),
            pltpu.VMEM((grp, h), jnp.float32),
            pltpu.SemaphoreType.DMA,
            pltpu.SemaphoreType.DMA,
            pltpu.SemaphoreType.DMA,
            pltpu.SemaphoreType.DMA,
        ],
    )
    def k2(c_hbm, idx_hbm, w_hbm, x_hbm, read_hbm, upd_hbm,
           idx_v, w_v, rows0, rows1, x0, x1, read_v, upd_v,
           sem0, sem1, xsem0, xsem1):
        nc = lax.axis_index("c")
        ns = lax.axis_index("s")
        wid = ns * 2 + nc
        base = wid * tpw
        pltpu.sync_copy(idx_hbm.at[wid], idx_v)
        rows = [rows0, rows1]
        sems = [sem0, sem1]
        xbuf = [x0, x1]
        xsems = [xsem0, xsem1]
        for b in range(2):
            pltpu.async_copy(c_hbm.at[idx_v.at[b]], rows[b], sems[b])
            pltpu.async_copy(x_hbm.at[pl.ds(base + b * grp, grp)], xbuf[b],
                             xsems[b])

        def pair_body(i, carry):
            g0 = i * 2
            for b in range(2):
                g = g0 + b
                tok0 = base + g * grp
                pltpu.make_async_copy(c_hbm.at[idx_v.at[g]], rows[b],
                                      sems[b]).wait()
                pltpu.make_async_copy(x_hbm.at[pl.ds(tok0, grp)], xbuf[b],
                                      xsems[b]).wait()
                pltpu.sync_copy(
                    w_hbm.at[wid, pl.ds(g * grp * 64, grp * 64)], w_v)
                rv = rows[b]
                xv = xbuf[b]

                def tok_body(t, carry2):
                    wk = [w_v[pl.ds(t * 64 + k * 16, 16)] for k in range(4)]
                    for hh in range(hc):
                        sl = pl.ds(hh * 16, 16)
                        r0 = rv[t * 4 + 0, sl]
                        r1 = rv[t * 4 + 1, sl]
                        r2 = rv[t * 4 + 2, sl]
                        r3 = rv[t * 4 + 3, sl]
                        read_v[t, sl] = (wk[0] * r0 + wk[1] * r1
                                         + wk[2] * r2 + wk[3] * r3)
                        upd_v[t, sl] = _EMA * (xv[t, sl] - r0)
                    return carry2

                lax.fori_loop(0, grp, tok_body, 0)
                pltpu.sync_copy(read_v, read_hbm.at[pl.ds(tok0, grp)])
                pltpu.sync_copy(upd_v, upd_hbm.at[pl.ds(tok0, grp)])

                @pl.when(g + 2 < n_grp)
                def _prefetch():
                    pltpu.async_copy(c_hbm.at[idx_v.at[g + 2]], rows[b],
                                     sems[b])
                    pltpu.async_copy(
                        x_hbm.at[pl.ds(tok0 + 2 * grp, grp)], xbuf[b],
                        xsems[b])
            return carry

        lax.fori_loop(0, n_grp // 2, pair_body, 0)

    return k2(contents, idx3, w3, xf)


def _proj_body(r_ref, w_ref, o_ref):
    o_ref[...] = lax.dot_general(r_ref[...], w_ref[...],
                                 (((1,), (1,)), ((), ())),
                                 preferred_element_type=jnp.float32)


def _out_proj(read, w_read, nt):
    n, h = read.shape
    return pl.pallas_call(
        _proj_body,
        grid=(n // nt,),
        in_specs=[
            pl.BlockSpec((nt, h), lambda i: (i, 0)),
            pl.BlockSpec((h, h), lambda i: (0, 0)),
        ],
        out_specs=pl.BlockSpec((nt, h), lambda i: (i, 0)),
        out_shape=jax.ShapeDtypeStruct((n, h), jnp.float32),
    )(read, w_read)


def _scatter_update_sc(contents, top1, upd, n_workers):
    m_, h = contents.shape
    n = top1.shape[0]
    tpv = n // 16
    hs = 128
    nhp = h // hs
    slots_c = m_ // 2
    rpt = slots_c // 16
    mesh = plsc.VectorSubcoreMesh(core_axis_name="c", subcore_axis_name="s")

    @functools.partial(
        pl.kernel,
        out_type=jax.ShapeDtypeStruct((m_, h), jnp.float32),
        mesh=mesh,
        scratch_types=[
            pltpu.VMEM((tpv,), jnp.int32),
            pltpu.VMEM((tpv,), jnp.int32),
            pltpu.VMEM((tpv // 2, hs), jnp.float32),
            pltpu.VMEM_SHARED((m_ // 2 + 16, hs), jnp.float32),
        ],
    )
    def k4(c_hbm, t1_hbm, u_hbm, o_hbm, idx_v, loc_v, uall, acc):
        nc = lax.axis_index("c")
        ns = lax.axis_index("s")
        base = ns * tpv
        row0 = ns * rpt
        lo = nc * slots_c
        pltpu.sync_copy(t1_hbm.at[pl.ds(base, tpv)], idx_v)

        def redir(j, carry):
            t1 = idx_v[pl.ds(j * 16, 16)]
            inb = (t1 >= lo) & (t1 < lo + slots_c)
            loc_v[pl.ds(j * 16, 16)] = jnp.where(inb, t1 - lo, slots_c)
            return carry

        lax.fori_loop(0, tpv // 16, redir, 0)

        def pass_body(hp, carry):
            hoff = hp * hs
            pltpu.sync_copy(
                c_hbm.at[pl.ds(lo + row0, rpt), pl.ds(hoff, hs)],
                acc.at[pl.ds(row0, rpt)])
            plsc.subcore_barrier()

            for half in range(2):
                toff = half * (tpv // 2)
                pltpu.sync_copy(
                    u_hbm.at[pl.ds(base + toff, tpv // 2), pl.ds(hoff, hs)],
                    uall)

                def win_body(j, carry2):
                    lv = loc_v[pl.ds(toff + j * 16, 16)]
                    pltpu.sync_copy(uall.at[pl.ds(j * 16, 16)], acc.at[lv],
                                    add=True)
                    return carry2

                lax.fori_loop(0, tpv // 32, win_body, 0)
            plsc.subcore_barrier()
            pltpu.sync_copy(acc.at[pl.ds(row0, rpt)],
                            o_hbm.at[pl.ds(lo + row0, rpt), pl.ds(hoff, hs)])
            plsc.subcore_barrier()
            return carry

        lax.fori_loop(0, nhp, pass_body, 0)

    return k4(contents, top1, upd)


def _scatter_body(c_ref, t1_ref, u_ref, o_ref):
    nn = pl.program_id(1)
    mt, nt = o_ref.shape[0], u_ref.shape[0]
    mloc = pl.program_id(0) * mt

    @pl.when(nn == 0)
    def _init():
        o_ref[...] = c_ref[...]

    idxv = t1_ref[0, 0, :]
    miota = lax.broadcasted_iota(jnp.int32, (mt, nt), 0) + mloc
    onehot = (miota == idxv[None, :]).astype(jnp.bfloat16)
    o_ref[...] += lax.dot_general(onehot, u_ref[...],
                                  (((1,), (0,)), ((), ())),
                                  preferred_element_type=jnp.float32)


def _scatter_update(contents, top1, upd_bf, mt, nt):
    m_, h = contents.shape
    n = top1.shape[0]
    t1r = top1.reshape(n // nt, 1, nt)
    grid = (m_ // mt, n // nt)
    return pl.pallas_call(
        _scatter_body,
        grid=grid,
        in_specs=[
            pl.BlockSpec((mt, h), lambda i, j: (i, 0)),
            pl.BlockSpec((1, 1, nt), lambda i, j: (j, 0, 0)),
            pl.BlockSpec((nt, h), lambda i, j: (j, 0)),
        ],
        out_specs=pl.BlockSpec((mt, h), lambda i, j: (i, 0)),
        out_shape=jax.ShapeDtypeStruct((m_, h), jnp.float32),
    )(contents, t1r, upd_bf)


def kernel(x, addresses, contents, W_addr, W_read):
    b, s, h = x.shape
    m_, _ = addresses.shape
    n = b * s
    xf = x.reshape(n, h)

    a_nb = _addr_norm(addresses, min(2048, m_))
    qb = _q_norm(xf, W_addr, min(1024, n))
    nt1 = min(1024, n)
    mt1 = min(1024, m_)
    ti4, w64 = _scores_topk(qb, a_nb, nt1, mt1)
    top_idx = ti4.T
    w = w64.T

    n_workers = 32
    grp = 8
    tpw = n // n_workers
    n_grp = tpw // grp
    idx3 = top_idx.reshape(n_workers, n_grp, grp * 4)
    w3 = w.reshape(n_workers, tpw * 64)
    read, upd = _gather_blend(contents, idx3, w3, xf, n_workers, grp, n_grp)

    out = _out_proj(read, W_read, min(1024, n)).reshape(b, s, h)

    new_contents = _scatter_update_sc(contents, ti4[0], upd, n_workers)
    return out, new_contents


# speedup vs baseline: 35.9551x; 1.0015x over previous
import functools

import jax
import jax.numpy as jnp
from jax import lax
from jax.experimental import pallas as pl
from jax.experimental.pallas import tpu as pltpu
from jax.experimental.pallas import tpu_sc as plsc

_EMA = 0.1
_EPS = 1e-08
_NEG_INF = float("-inf")
_BIG_I32 = 2 ** 30


def _anorm_body(a_ref, o_ref):
    a = a_ref[...]
    an = jnp.sqrt(jnp.sum(a * a, axis=1, keepdims=True))
    o_ref[...] = (a / jnp.maximum(an, _EPS)).astype(jnp.bfloat16)


def _addr_norm(addresses, mt):
    m_, h = addresses.shape
    return pl.pallas_call(
        _anorm_body,
        grid=(m_ // mt,),
        in_specs=[pl.BlockSpec((mt, h), lambda i: (i, 0))],
        out_specs=pl.BlockSpec((mt, h), lambda i: (i, 0)),
        out_shape=jax.ShapeDtypeStruct((m_, h), jnp.bfloat16),
    )(addresses)


def _extract4(s, bm, mt, nt, bv_scr, bi_scr):
    iota_l = lax.broadcasted_iota(jnp.int32, (mt, nt), 0)
    bvs, bis = [], []
    for k in range(4):
        mx = jnp.max(s, axis=0, keepdims=True)
        ix = jnp.min(jnp.where(s == mx, iota_l, _BIG_I32), axis=0,
                     keepdims=True)
        bvs.append(mx)
        bis.append(ix)
        if k < 3:
            s = jnp.where(iota_l == ix, _NEG_INF, s)
    pad_v = jnp.full((4, nt), _NEG_INF, jnp.float32)
    pad_i = jnp.full((4, nt), _BIG_I32, jnp.int32)
    off = pl.multiple_of(bm * 8 + 8, 8)
    bv_scr[pl.ds(off, 8), :] = jnp.concatenate(bvs + [pad_v], axis=0)
    bi_scr[pl.ds(off, 8), :] = jnp.concatenate(
        [b + bm * mt for b in bis] + [pad_i], axis=0)


def _qnorm_body(x_ref, wa_ref, o_ref):
    q = lax.dot_general(x_ref[...], wa_ref[...], (((1,), (1,)), ((), ())),
                        preferred_element_type=jnp.float32)
    qn = jnp.sqrt(jnp.sum(q * q, axis=1, keepdims=True))
    o_ref[...] = (q / jnp.maximum(qn, _EPS)).astype(jnp.bfloat16)


def _q_norm(xf, w_addr, nt):
    n, h = xf.shape
    return pl.pallas_call(
        _qnorm_body,
        grid=(n // nt,),
        in_specs=[
            pl.BlockSpec((nt, h), lambda i: (i, 0)),
            pl.BlockSpec((h, h), lambda i: (0, 0)),
        ],
        out_specs=pl.BlockSpec((nt, h), lambda i: (i, 0)),
        out_shape=jax.ShapeDtypeStruct((n, h), jnp.bfloat16),
    )(xf, w_addr)


def _topk_body(qb_ref, a_ref, idx_ref, w_ref, s_scr, bv_scr, bi_scr):
    m = pl.program_id(1)
    n_m = pl.num_programs(1)
    nt, mt = qb_ref.shape[0], a_ref.shape[0]

    s = lax.dot_general(a_ref[...], qb_ref[...], (((1,), (1,)), ((), ())),
                        preferred_element_type=jnp.float32)

    sprev = s_scr[...]
    _extract4(sprev, m - 1, mt, nt, bv_scr, bi_scr)
    s_scr[...] = s

    @pl.when(m == n_m - 1)
    def _final():
        _extract4(s, n_m - 1, mt, nt, bv_scr, bi_scr)
        cv = bv_scr[pl.ds(8, 8 * n_m), :]
        ci = bi_scr[pl.ds(8, 8 * n_m), :]
        nvs, nis = [], []
        for _ in range(4):
            mx = jnp.max(cv, axis=0, keepdims=True)
            ix = jnp.min(jnp.where(cv == mx, ci, _BIG_I32), axis=0,
                         keepdims=True)
            nvs.append(mx)
            nis.append(ix)
            cv = jnp.where(ci == ix, _NEG_INF, cv)
        tv = jnp.concatenate(nvs, axis=0)
        e = jnp.exp(tv - jnp.max(tv, axis=0, keepdims=True))
        wsm = e / jnp.sum(e, axis=0, keepdims=True)
        w_ref[...] = jnp.broadcast_to(wsm[:, None, :],
                                      (4, 16, nt)).reshape(64, nt)
        idx_ref[...] = jnp.concatenate(nis, axis=0)


def _scores_topk(qb, addresses, nt, mt):
    n, h = qb.shape
    m_, _ = addresses.shape
    grid = (n // nt, m_ // mt)
    return pl.pallas_call(
        _topk_body,
        grid=grid,
        in_specs=[
            pl.BlockSpec((nt, h), lambda i, j: (i, 0)),
            pl.BlockSpec((mt, h), lambda i, j: (j, 0)),
        ],
        out_specs=[
            pl.BlockSpec((4, nt), lambda i, j: (0, i)),
            pl.BlockSpec((64, nt), lambda i, j: (0, i)),
        ],
        out_shape=[
            jax.ShapeDtypeStruct((4, n), jnp.int32),
            jax.ShapeDtypeStruct((64, n), jnp.float32),
        ],
        scratch_shapes=[
            pltpu.VMEM((mt, nt), jnp.float32),
            pltpu.VMEM((8 * (m_ // mt) + 8, nt), jnp.float32),
            pltpu.VMEM((8 * (m_ // mt) + 8, nt), jnp.int32),
        ],
    )(qb, addresses)


def _gather_blend(contents, idx3, w3, xf, n_workers, grp, n_grp):
    m_, h = contents.shape
    n, _ = xf.shape
    tpw = n // n_workers
    hc = h // 16
    mesh = plsc.VectorSubcoreMesh(core_axis_name="c", subcore_axis_name="s")

    @functools.partial(
        pl.kernel,
        out_type=[
            jax.ShapeDtypeStruct((n, h), jnp.float32),
            jax.ShapeDtypeStruct((n, h), jnp.float32),
        ],
        mesh=mesh,
        scratch_types=[
            pltpu.VMEM((n_grp, grp * 4), jnp.int32),
            pltpu.VMEM((grp * 64,), jnp.float32),
            pltpu.VMEM((grp * 4, h), jnp.float32),
            pltpu.VMEM((grp * 4, h), jnp.float32),
            pltpu.VMEM((grp, h), jnp.float32),
            pltpu.VMEM((grp, h), jnp.float32),
            pltpu.VMEM((grp, h), jnp.float32),
            pltpu.VMEM((grp, h), jnp.float32),
            pltpu.SemaphoreType.DMA,
            pltpu.SemaphoreType.DMA,
            pltpu.SemaphoreType.DMA,
            pltpu.SemaphoreType.DMA,
        ],
    )
    def k2(c_hbm, idx_hbm, w_hbm, x_hbm, read_hbm, upd_hbm,
           idx_v, w_v, rows0, rows1, x0, x1, read_v, upd_v,
           sem0, sem1, xsem0, xsem1):
        nc = lax.axis_index("c")
        ns = lax.axis_index("s")
        wid = ns * 2 + nc
        base = wid * tpw
        pltpu.sync_copy(idx_hbm.at[wid], idx_v)
        rows = [rows0, rows1]
        sems = [sem0, sem1]
        xbuf = [x0, x1]
        xsems = [xsem0, xsem1]
        for b in range(2):
            pltpu.async_copy(c_hbm.at[idx_v.at[b]], rows[b], sems[b])
            pltpu.async_copy(x_hbm.at[pl.ds(base + b * grp, grp)], xbuf[b],
                             xsems[b])

        def pair_body(i, carry):
            g0 = i * 2
            for b in range(2):
                g = g0 + b
                tok0 = base + g * grp
                pltpu.make_async_copy(c_hbm.at[idx_v.at[g]], rows[b],
                                      sems[b]).wait()
                pltpu.make_async_copy(x_hbm.at[pl.ds(tok0, grp)], xbuf[b],
                                      xsems[b]).wait()
                pltpu.sync_copy(
                    w_hbm.at[wid, pl.ds(g * grp * 64, grp * 64)], w_v)
                rv = rows[b]
                xv = xbuf[b]

                def tok_body(t, carry2):
                    wk = [w_v[pl.ds(t * 64 + k * 16, 16)] for k in range(4)]
                    for hh in range(hc):
                        sl = pl.ds(hh * 16, 16)
                        r0 = rv[t * 4 + 0, sl]
                        r1 = rv[t * 4 + 1, sl]
                        r2 = rv[t * 4 + 2, sl]
                        r3 = rv[t * 4 + 3, sl]
                        read_v[t, sl] = (wk[0] * r0 + wk[1] * r1
                                         + wk[2] * r2 + wk[3] * r3)
                        upd_v[t, sl] = _EMA * (xv[t, sl] - r0)
                    return carry2

                lax.fori_loop(0, grp, tok_body, 0)
                pltpu.sync_copy(read_v, read_hbm.at[pl.ds(tok0, grp)])
                pltpu.sync_copy(upd_v, upd_hbm.at[pl.ds(tok0, grp)])

                @pl.when(g + 2 < n_grp)
                def _prefetch():
                    pltpu.async_copy(c_hbm.at[idx_v.at[g + 2]], rows[b],
                                     sems[b])
                    pltpu.async_copy(
                        x_hbm.at[pl.ds(tok0 + 2 * grp, grp)], xbuf[b],
                        xsems[b])
            return carry

        lax.fori_loop(0, n_grp // 2, pair_body, 0)

    return k2(contents, idx3, w3, xf)


def _proj_body(r_ref, w_ref, o_ref):
    o_ref[...] = lax.dot_general(r_ref[...], w_ref[...],
                                 (((1,), (1,)), ((), ())),
                                 preferred_element_type=jnp.float32)


def _out_proj(read, w_read, nt):
    n, h = read.shape
    return pl.pallas_call(
        _proj_body,
        grid=(n // nt,),
        in_specs=[
            pl.BlockSpec((nt, h), lambda i: (i, 0)),
            pl.BlockSpec((h, h), lambda i: (0, 0)),
        ],
        out_specs=pl.BlockSpec((nt, h), lambda i: (i, 0)),
        out_shape=jax.ShapeDtypeStruct((n, h), jnp.float32),
    )(read, w_read)


def _scatter_update_sc(contents, top1, upd, n_workers):
    m_, h = contents.shape
    n = top1.shape[0]
    tpv = n // 16
    hs = 128
    nhp = h // hs
    slots_c = m_ // 2
    rpt = slots_c // 16
    mesh = plsc.VectorSubcoreMesh(core_axis_name="c", subcore_axis_name="s")

    @functools.partial(
        pl.kernel,
        out_type=jax.ShapeDtypeStruct((m_, h), jnp.float32),
        mesh=mesh,
        scratch_types=[
            pltpu.VMEM((tpv,), jnp.int32),
            pltpu.VMEM((tpv,), jnp.int32),
            pltpu.VMEM((tpv // 2, hs), jnp.float32),
            pltpu.VMEM_SHARED((m_ // 2 + 16, hs), jnp.float32),
        ],
    )
    def k4(c_hbm, t1_hbm, u_hbm, o_hbm, idx_v, loc_v, uall, acc):
        nc = lax.axis_index("c")
        ns = lax.axis_index("s")
        base = ns * tpv
        row0 = ns * rpt
        lo = nc * slots_c
        pltpu.sync_copy(t1_hbm.at[pl.ds(base, tpv)], idx_v)

        def redir(j, carry):
            t1 = idx_v[pl.ds(j * 16, 16)]
            inb = (t1 >= lo) & (t1 < lo + slots_c)
            loc_v[pl.ds(j * 16, 16)] = jnp.where(inb, t1 - lo, slots_c)
            return carry

        lax.fori_loop(0, tpv // 16, redir, 0)

        def pass_body(hp, carry):
            hoff = hp * hs
            pltpu.sync_copy(
                c_hbm.at[pl.ds(lo + row0, rpt), pl.ds(hoff, hs)],
                acc.at[pl.ds(row0, rpt)])
            plsc.subcore_barrier()

            for half in range(2):
                toff = half * (tpv // 2)
                pltpu.sync_copy(
                    u_hbm.at[pl.ds(base + toff, tpv // 2), pl.ds(hoff, hs)],
                    uall)

                def win_body(j, carry2):
                    lv = loc_v[pl.ds(toff + j * 16, 16)]
                    pltpu.sync_copy(uall.at[pl.ds(j * 16, 16)], acc.at[lv],
                                    add=True)
                    return carry2

                lax.fori_loop(0, tpv // 32, win_body, 0)
            plsc.subcore_barrier()
            pltpu.sync_copy(acc.at[pl.ds(row0, rpt)],
                            o_hbm.at[pl.ds(lo + row0, rpt), pl.ds(hoff, hs)])
            plsc.subcore_barrier()
            return carry

        lax.fori_loop(0, nhp, pass_body, 0)

    return k4(contents, top1, upd)


def kernel(x, addresses, contents, W_addr, W_read):
    b, s, h = x.shape
    m_, _ = addresses.shape
    n = b * s
    xf = x.reshape(n, h)

    a_nb = _addr_norm(addresses, min(2048, m_))
    qb = _q_norm(xf, W_addr, min(1024, n))
    nt1 = min(1024, n)
    mt1 = min(1024, m_)
    ti4, w64 = _scores_topk(qb, a_nb, nt1, mt1)
    top_idx = ti4.T
    w = w64.T

    n_workers = 32
    grp = 8
    tpw = n // n_workers
    n_grp = tpw // grp
    idx3 = top_idx.reshape(n_workers, n_grp, grp * 4)
    w3 = w.reshape(n_workers, tpw * 64)
    read, upd = _gather_blend(contents, idx3, w3, xf, n_workers, grp, n_grp)

    out = _out_proj(read, W_read, min(1024, n)).reshape(b, s, h)

    new_contents = _scatter_update_sc(contents, ti4[0], upd, n_workers)
    return out, new_contents
